# Initial kernel scaffold; baseline (speedup 1.0000x reference)
import jax
import jax.numpy as jnp
from jax.experimental import pallas as pl


def kernel(X, vertex_idx, hyperedge_idx, W1, b1, W2, b2, W3, b3, A1, ab1, A2):
    raise NotImplementedError("write your pallas kernel here")


# speedup vs baseline: 1.3467x; 1.3467x over previous
import functools

import jax
import jax.numpy as jnp
from jax import lax
from jax.experimental import pallas as pl
from jax.experimental.pallas import tpu as pltpu
from jax.experimental.pallas import tpu_sc as plsc


def _mm_bias_act(x, w, b, act, block_rows=1000):
    n, k = x.shape
    ko, m = w.shape
    assert k == ko
    assert n % block_rows == 0

    def body(x_ref, w_ref, b_ref, o_ref):
        acc = jnp.dot(x_ref[...], w_ref[...], preferred_element_type=jnp.float32)
        acc = acc + b_ref[...]
        o_ref[...] = act(acc)

    return pl.pallas_call(
        body,
        grid=(n // block_rows,),
        in_specs=[
            pl.BlockSpec((block_rows, k), lambda i: (i, 0)),
            pl.BlockSpec((k, m), lambda i: (0, 0)),
            pl.BlockSpec((1, m), lambda i: (0, 0)),
        ],
        out_specs=pl.BlockSpec((block_rows, m), lambda i: (i, 0)),
        out_shape=jax.ShapeDtypeStruct((n, m), jnp.float32),
    )(x, w, b.reshape(1, m))


def _decode_kernel(xl, z, block_rows=1000):
    n, c = xl.shape
    e, c2 = z.shape
    assert c == c2

    def body(x_ref, z_ref, o_ref):
        acc = jax.lax.dot_general(
            x_ref[...], z_ref[...],
            dimension_numbers=(((1,), (1,)), ((), ())),
            preferred_element_type=jnp.float32)
        o_ref[...] = jax.nn.sigmoid(acc)

    return pl.pallas_call(
        body,
        grid=(n // block_rows,),
        in_specs=[
            pl.BlockSpec((block_rows, c), lambda i: (i, 0)),
            pl.BlockSpec((e, c), lambda i: (0, 0)),
        ],
        out_specs=pl.BlockSpec((block_rows, e), lambda i: (i, 0)),
        out_shape=jax.ShapeDtypeStruct((n, e), jnp.float32),
    )(xl, z)


def _scores_kernel(xl, a1, ab1, a2, block_rows=1000):
    n, c = xl.shape
    c2, ah = a1.shape
    assert c == c2

    def body(x_ref, a1_ref, ab1_ref, a2_ref, o_ref):
        t = jnp.tanh(
            jnp.dot(x_ref[...], a1_ref[...], preferred_element_type=jnp.float32)
            + ab1_ref[...])
        o_ref[...] = jnp.dot(t, a2_ref[...], preferred_element_type=jnp.float32)

    return pl.pallas_call(
        body,
        grid=(n // block_rows,),
        in_specs=[
            pl.BlockSpec((block_rows, c), lambda i: (i, 0)),
            pl.BlockSpec((c, ah), lambda i: (0, 0)),
            pl.BlockSpec((1, ah), lambda i: (0, 0)),
            pl.BlockSpec((ah, 1), lambda i: (0, 0)),
        ],
        out_specs=pl.BlockSpec((block_rows, 1), lambda i: (i, 0)),
        out_shape=jax.ShapeDtypeStruct((n, 1), jnp.float32),
    )(xl, a1, ab1.reshape(1, ah), a2)


def kernel(X, vertex_idx, hyperedge_idx, W1, b1, W2, b2, W3, b3, A1, ab1, A2):
    n, c_in = X.shape
    nnz = vertex_idx.shape[0]
    e_he = 2000
    c_out = W2.shape[1]

    ones = jnp.ones((nnz,), jnp.float32)
    dv = jax.ops.segment_sum(ones, vertex_idx, n)
    de = jax.ops.segment_sum(ones, hyperedge_idx, e_he)
    dv_is = jnp.where(dv > 0, lax.rsqrt(jnp.maximum(dv, 1e-12)), 0.0)
    de_inv = jnp.where(de > 0, 1.0 / jnp.maximum(de, 1e-12), 0.0)

    def smooth(h):
        m = (h * dv_is[:, None])[vertex_idx]
        y = jax.ops.segment_sum(m, hyperedge_idx, e_he) * de_inv[:, None]
        return jax.ops.segment_sum(y[hyperedge_idx], vertex_idx, n) * dv_is[:, None]

    h0 = _mm_bias_act(X, W1, b1, jax.nn.relu)
    h1 = smooth(h0)

    wc = jnp.concatenate([W2, W3], axis=1)
    bc = jnp.concatenate([b2, b3], axis=0)
    g = _mm_bias_act(h1, wc, bc, lambda x: x)
    s2 = smooth(g)
    mu = s2[:, :c_out]
    logvar = s2[:, c_out:]
    xl = mu

    scores = _scores_kernel(xl, A1, ab1, A2)[:, 0]
    gmax = jnp.max(scores)
    s = scores[vertex_idx]
    ex = jnp.exp(s - gmax)
    den = jax.ops.segment_sum(ex, hyperedge_idx, e_he)
    beta = ex / jnp.maximum(den[hyperedge_idx], 1e-12)
    z = jax.ops.segment_sum(beta[:, None] * xl[vertex_idx], hyperedge_idx, e_he)

    h_out = _decode_kernel(xl, z)
    return (xl, z, h_out, mu, logvar, beta)


# speedup vs baseline: 6.4791x; 4.8112x over previous
import functools

import jax
import jax.numpy as jnp
from jax import lax
from jax.experimental import pallas as pl
from jax.experimental.pallas import tpu as pltpu
from jax.experimental.pallas import tpu_sc as plsc

_NC = 2
_NS = 16
_L = 16
_NW = _NC * _NS

_N = 10000
_E = 2000
_NNZ = 320000

_CHA = 80
_CHT = 128
_PT = _CHA * _CHT
_NNZP = _PT * _NW
_ROWS = _NNZP // _CHA


def _sc_mesh():
    return plsc.VectorSubcoreMesh(
        core_axis_name="c", subcore_axis_name="s",
        num_cores=_NC, num_subcores=_NS)


_SC_PARAMS = pltpu.CompilerParams(
    use_tc_tiling_on_sc=False, needs_layout_passes=False)


def _sc_degrees(v_s, e_s, ones_tab, zv, ze):
    @functools.partial(
        pl.kernel,
        out_type=(jax.ShapeDtypeStruct((_NC, _N + 1, _L), jnp.float32),
                  jax.ShapeDtypeStruct((_NC, _E + 1, _L), jnp.float32)),
        mesh=_sc_mesh(),
        compiler_params=_SC_PARAMS,
        scratch_types=[
            pltpu.VMEM((_CHT, _CHA), jnp.int32),
            pltpu.VMEM((_CHT, _CHA), jnp.int32),
            pltpu.VMEM((_CHA, _L), jnp.float32),
            pltpu.VMEM_SHARED((_N + 1, _L), jnp.float32),
            pltpu.VMEM_SHARED((_E + 1, _L), jnp.float32),
        ],
    )
    def kfn(v2_h, e2_h, ones_h, zv_h, ze_h, dvp_h, dep_h,
            vbuf, ebuf, ones_v, accv, acce):
        c = lax.axis_index("c")
        s = lax.axis_index("s")
        wid = c * _NS + s
        pltpu.sync_copy(v2_h.at[pl.ds(wid * _CHT, _CHT)], vbuf)
        pltpu.sync_copy(e2_h.at[pl.ds(wid * _CHT, _CHT)], ebuf)
        pltpu.sync_copy(ones_h, ones_v)

        @pl.when(s == 0)
        def _():
            pltpu.sync_copy(zv_h, accv)
            pltpu.sync_copy(ze_h, acce)
        plsc.subcore_barrier()

        @pl.loop(0, _CHT)
        def _(j):
            pltpu.sync_copy(ones_v, accv.at[vbuf.at[j]], add=True)
            pltpu.sync_copy(ones_v, acce.at[ebuf.at[j]], add=True)

        plsc.subcore_barrier()

        @pl.when(s == 0)
        def _():
            pltpu.sync_copy(accv, dvp_h.at[c])
            pltpu.sync_copy(acce, dep_h.at[c])

    return kfn(v_s, e_s, ones_tab, zv, ze)


def _sc_pair_pass(table, g2, s2, out_rows, zeros_acc):
    _, w = table.shape

    @functools.partial(
        pl.kernel,
        out_type=jax.ShapeDtypeStruct((_NC, out_rows + 1, w), jnp.float32),
        mesh=_sc_mesh(),
        compiler_params=_SC_PARAMS,
        scratch_types=[
            pltpu.VMEM((_CHT, _CHA), jnp.int32),
            pltpu.VMEM((_CHT, _CHA), jnp.int32),
            pltpu.VMEM((_CHA, w), jnp.float32),
            pltpu.VMEM((_CHA, w), jnp.float32),
            pltpu.VMEM_SHARED((out_rows + 1, w), jnp.float32),
            pltpu.SemaphoreType.DMA,
            pltpu.SemaphoreType.DMA,
        ],
    )
    def kfn(tab_h, g2_h, s2_h, zz_h, out_h,
            gbuf, sbuf, rows0, rows1, acc, sem0, sem1):
        c = lax.axis_index("c")
        s = lax.axis_index("s")
        wid = c * _NS + s
        pltpu.sync_copy(g2_h.at[pl.ds(wid * _CHT, _CHT)], gbuf)
        pltpu.sync_copy(s2_h.at[pl.ds(wid * _CHT, _CHT)], sbuf)

        @pl.when(s == 0)
        def _():
            pltpu.sync_copy(zz_h, acc)
        plsc.subcore_barrier()

        rows = (rows0, rows1)
        sems = (sem0, sem1)
        pltpu.async_copy(tab_h.at[gbuf.at[0]], rows0, sem0)
        pltpu.async_copy(tab_h.at[gbuf.at[1]], rows1, sem1)

        @pl.loop(0, _CHT, step=2)
        def _(j):
            for b in range(2):
                k = j + b
                pltpu.make_async_copy(
                    tab_h.at[gbuf.at[k]], rows[b], sems[b]).wait()
                pltpu.sync_copy(rows[b], acc.at[sbuf.at[k]], add=True)

                @pl.when(k + 2 < _CHT)
                def _():
                    pltpu.async_copy(
                        tab_h.at[gbuf.at[k + 2]], rows[b], sems[b])

        plsc.subcore_barrier()

        @pl.when(s == 0)
        def _():
            pltpu.sync_copy(acc, out_h.at[c])

    return kfn(table, g2, s2, zeros_acc)


def _sc_att_ex(ssh, v_g, e_s, zrow, zden):
    @functools.partial(
        pl.kernel,
        out_type=(jax.ShapeDtypeStruct((_NNZP,), jnp.float32),
                  jax.ShapeDtypeStruct((_NC, _E + 1, _L), jnp.float32)),
        mesh=_sc_mesh(),
        compiler_params=_SC_PARAMS,
        scratch_types=[
            pltpu.VMEM((_CHT, _CHA), jnp.int32),
            pltpu.VMEM((_CHT, _CHA), jnp.int32),
            pltpu.VMEM((_N,), jnp.float32),
            pltpu.VMEM((_PT,), jnp.float32),
            pltpu.VMEM((_CHA, _L), jnp.float32),
            pltpu.VMEM_SHARED((_E + 1, _L), jnp.float32),
        ],
    )
    def kfn(ssh_h, v2_h, e2_h, zrow_h, zden_h, ex_h, denp_h,
            vbuf, ebuf, stab, extile, exbuf, accd):
        c = lax.axis_index("c")
        s = lax.axis_index("s")
        wid = c * _NS + s
        pltpu.sync_copy(v2_h.at[pl.ds(wid * _CHT, _CHT)], vbuf)
        pltpu.sync_copy(e2_h.at[pl.ds(wid * _CHT, _CHT)], ebuf)
        pltpu.sync_copy(ssh_h, stab)
        pltpu.sync_copy(zrow_h, exbuf)

        @pl.when(s == 0)
        def _():
            pltpu.sync_copy(zden_h, accd)
        plsc.subcore_barrier()

        zeros16 = jnp.zeros((_L,), jnp.int32)

        @pl.loop(0, _CHT)
        def _(k):
            for u in range(_CHA // _L):
                idx = vbuf[k, pl.ds(u * _L, _L)]
                ex = jnp.exp(plsc.load_gather(stab, [idx]))
                extile[pl.ds(k * _CHA + u * _L, _L)] = ex
                rowi = lax.iota(jnp.int32, _L) + (u * _L)
                plsc.store_scatter(exbuf, [rowi, zeros16], ex)
            pltpu.sync_copy(exbuf, accd.at[ebuf.at[k]], add=True)

        plsc.subcore_barrier()
        pltpu.sync_copy(extile, ex_h.at[pl.ds(wid * _PT, _PT)])

        @pl.when(s == 0)
        def _():
            pltpu.sync_copy(accd, denp_h.at[c])

    return kfn(ssh, v_g, e_s, zrow, zden)


def _sc_att_z(ex, rden, mu, v_g, e_g, e_s, zz):
    c_out = mu.shape[1]

    @functools.partial(
        pl.kernel,
        out_type=(jax.ShapeDtypeStruct((_NNZP,), jnp.float32),
                  jax.ShapeDtypeStruct((_NC, _E + 1, c_out), jnp.float32)),
        mesh=_sc_mesh(),
        compiler_params=_SC_PARAMS,
        scratch_types=[
            pltpu.VMEM((_CHT, _CHA), jnp.int32),
            pltpu.VMEM((_CHT, _CHA), jnp.int32),
            pltpu.VMEM((_CHT, _CHA), jnp.int32),
            pltpu.VMEM((_E,), jnp.float32),
            pltpu.VMEM((_PT,), jnp.float32),
            pltpu.VMEM((_PT,), jnp.float32),
            pltpu.VMEM((_CHA, c_out), jnp.float32),
            pltpu.VMEM((_CHA, c_out), jnp.float32),
            pltpu.VMEM_SHARED((_E + 1, c_out), jnp.float32),
            pltpu.SemaphoreType.DMA,
            pltpu.SemaphoreType.DMA,
        ],
    )
    def kfn(ex_h, rden_h, mu_h, vg_h, eg_h, es_h, zz_h, beta_h, zp_h,
            vbuf, egbuf, esbuf, rdtab, extile, betatile, rows0, rows1,
            accz, sem0, sem1):
        c = lax.axis_index("c")
        s = lax.axis_index("s")
        wid = c * _NS + s
        pltpu.sync_copy(vg_h.at[pl.ds(wid * _CHT, _CHT)], vbuf)
        pltpu.sync_copy(eg_h.at[pl.ds(wid * _CHT, _CHT)], egbuf)
        pltpu.sync_copy(es_h.at[pl.ds(wid * _CHT, _CHT)], esbuf)
        pltpu.sync_copy(rden_h, rdtab)
        pltpu.sync_copy(ex_h.at[pl.ds(wid * _PT, _PT)], extile)

        @pl.when(s == 0)
        def _():
            pltpu.sync_copy(zz_h, accz)
        plsc.subcore_barrier()

        rows = (rows0, rows1)
        sems = (sem0, sem1)
        pltpu.async_copy(mu_h.at[vbuf.at[0]], rows0, sem0)
        pltpu.async_copy(mu_h.at[vbuf.at[1]], rows1, sem1)

        @pl.loop(0, _CHT, step=2)
        def _(j):
            for b in range(2):
                k = j + b
                pltpu.make_async_copy(
                    mu_h.at[vbuf.at[k]], rows[b], sems[b]).wait()
                for u in range(_CHA // _L):
                    eidx = egbuf[k, pl.ds(u * _L, _L)]
                    rd = plsc.load_gather(rdtab, [eidx])
                    exv = extile[pl.ds(k * _CHA + u * _L, _L)]
                    bet = exv * rd
                    betatile[pl.ds(k * _CHA + u * _L, _L)] = bet
                    rowi = lax.iota(jnp.int32, _L) + (u * _L)
                    for col in range(c_out):
                        ci = jnp.full((_L,), col, jnp.int32)
                        vals = plsc.load_gather(rows[b], [rowi, ci])
                        plsc.store_scatter(rows[b], [rowi, ci], vals * bet)
                pltpu.sync_copy(rows[b], accz.at[esbuf.at[k]], add=True)

                @pl.when(k + 2 < _CHT)
                def _():
                    pltpu.async_copy(
                        mu_h.at[vbuf.at[k + 2]], rows[b], sems[b])

        plsc.subcore_barrier()
        pltpu.sync_copy(betatile, beta_h.at[pl.ds(wid * _PT, _PT)])

        @pl.when(s == 0)
        def _():
            pltpu.sync_copy(accz, zp_h.at[c])

    return kfn(ex, rden, mu, v_g, e_g, e_s, zz)


def _mm_relu(x, w, b, block_rows=1000):
    n, k = x.shape
    _, m = w.shape

    def body(x_ref, w_ref, b_ref, o_ref):
        acc = jnp.dot(x_ref[...], w_ref[...], preferred_element_type=jnp.float32)
        o_ref[...] = jax.nn.relu(acc + b_ref[...])

    return pl.pallas_call(
        body,
        grid=(n // block_rows,),
        in_specs=[
            pl.BlockSpec((block_rows, k), lambda i: (i, 0)),
            pl.BlockSpec((k, m), lambda i: (0, 0)),
            pl.BlockSpec((1, m), lambda i: (0, 0)),
        ],
        out_specs=pl.BlockSpec((block_rows, m), lambda i: (i, 0)),
        out_shape=jax.ShapeDtypeStruct((n, m), jnp.float32),
    )(x, w, b.reshape(1, m))


def _deg_combine_scale(dvp, dep, h0, block_rows=1000):
    n = h0.shape[0]
    c = h0.shape[1]

    def body(dv_ref, de_ref, h_ref, hs_ref, dvis_ref, dei_ref):
        i = pl.program_id(0)
        dv = dv_ref[0, :, 0:1] + dv_ref[1, :, 0:1]
        dvis = jnp.where(dv > 0, lax.rsqrt(jnp.maximum(dv, 1e-12)), 0.0)
        dvis_ref[...] = dvis
        hs_ref[...] = h_ref[...] * dvis

        @pl.when(i == 0)
        def _():
            de = de_ref[0, :, 0:1] + de_ref[1, :, 0:1]
            dei_ref[...] = jnp.where(de > 0, 1.0 / jnp.maximum(de, 1e-12), 0.0)

    return pl.pallas_call(
        body,
        grid=(n // block_rows,),
        in_specs=[
            pl.BlockSpec((2, block_rows, _L), lambda i: (0, i, 0)),
            pl.BlockSpec((2, _E, _L), lambda i: (0, 0, 0)),
            pl.BlockSpec((block_rows, c), lambda i: (i, 0)),
        ],
        out_specs=[
            pl.BlockSpec((block_rows, c), lambda i: (i, 0)),
            pl.BlockSpec((block_rows, 1), lambda i: (i, 0)),
            pl.BlockSpec((_E, 1), lambda i: (0, 0)),
        ],
        out_shape=[
            jax.ShapeDtypeStruct((n, c), jnp.float32),
            jax.ShapeDtypeStruct((n, 1), jnp.float32),
            jax.ShapeDtypeStruct((_E, 1), jnp.float32),
        ],
    )(dvp, dep, h0)


def _comb_scale(yp, scale, rows):
    _, _, w = yp.shape

    def body(y_ref, s_ref, o_ref):
        o_ref[...] = (y_ref[0] + y_ref[1]) * s_ref[...]

    return pl.pallas_call(
        body,
        grid=(1,),
        in_specs=[
            pl.BlockSpec((2, rows, w), lambda i: (0, 0, 0)),
            pl.BlockSpec((rows, 1), lambda i: (0, 0)),
        ],
        out_specs=pl.BlockSpec((rows, w), lambda i: (0, 0)),
        out_shape=jax.ShapeDtypeStruct((rows, w), jnp.float32),
    )(yp, scale)


def _smooth_out_project(ppa, ppb, dv_is, wc, bc, block_rows=1000):
    half = ppa.shape[2]
    n = dv_is.shape[0]
    m = wc.shape[1]

    def body(pa_ref, pb_ref, d_ref, w_ref, b_ref, o_ref):
        d = d_ref[...]
        h1a = (pa_ref[0] + pa_ref[1]) * d
        h1b = (pb_ref[0] + pb_ref[1]) * d
        g = (jnp.dot(h1a, w_ref[0:half], preferred_element_type=jnp.float32)
             + jnp.dot(h1b, w_ref[half:], preferred_element_type=jnp.float32))
        o_ref[...] = (g + b_ref[...]) * d

    return pl.pallas_call(
        body,
        grid=(n // block_rows,),
        in_specs=[
            pl.BlockSpec((2, block_rows, half), lambda i: (0, i, 0)),
            pl.BlockSpec((2, block_rows, half), lambda i: (0, i, 0)),
            pl.BlockSpec((block_rows, 1), lambda i: (i, 0)),
            pl.BlockSpec((2 * half, m), lambda i: (0, 0)),
            pl.BlockSpec((1, m), lambda i: (0, 0)),
        ],
        out_specs=pl.BlockSpec((block_rows, m), lambda i: (i, 0)),
        out_shape=jax.ShapeDtypeStruct((n, m), jnp.float32),
    )(ppa, ppb, dv_is, wc, bc.reshape(1, m))


def _final_smooth_attention(rp, dv_is, a1, ab1, a2, c_out, block_rows=1000):
    _, _, w = rp.shape
    n = dv_is.shape[0]
    ah = a1.shape[1]

    def body(r_ref, d_ref, a1_ref, ab1_ref, a2_ref,
             mu_ref, lv_ref, sc_ref, mx_ref):
        i = pl.program_id(0)
        s2 = (r_ref[0] + r_ref[1]) * d_ref[...]
        mu = s2[:, :c_out]
        mu_ref[...] = mu
        lv_ref[...] = s2[:, c_out:]
        t = jnp.tanh(jnp.dot(mu, a1_ref[...], preferred_element_type=jnp.float32)
                     + ab1_ref[...])
        sc = jnp.dot(t, a2_ref[...], preferred_element_type=jnp.float32)
        sc_ref[...] = sc

        @pl.when(i == 0)
        def _():
            mx_ref[...] = jnp.full_like(mx_ref[...], -jnp.inf)
        mx_ref[...] = jnp.maximum(mx_ref[...], jnp.max(sc))

    return pl.pallas_call(
        body,
        grid=(n // block_rows,),
        in_specs=[
            pl.BlockSpec((2, block_rows, w), lambda i: (0, i, 0)),
            pl.BlockSpec((block_rows, 1), lambda i: (i, 0)),
            pl.BlockSpec((c_out, ah), lambda i: (0, 0)),
            pl.BlockSpec((1, ah), lambda i: (0, 0)),
            pl.BlockSpec((ah, 1), lambda i: (0, 0)),
        ],
        out_specs=[
            pl.BlockSpec((block_rows, c_out), lambda i: (i, 0)),
            pl.BlockSpec((block_rows, c_out), lambda i: (i, 0)),
            pl.BlockSpec((block_rows, 1), lambda i: (i, 0)),
            pl.BlockSpec((1, 1), lambda i: (0, 0)),
        ],
        out_shape=[
            jax.ShapeDtypeStruct((n, c_out), jnp.float32),
            jax.ShapeDtypeStruct((n, c_out), jnp.float32),
            jax.ShapeDtypeStruct((n, 1), jnp.float32),
            jax.ShapeDtypeStruct((1, 1), jnp.float32),
        ],
    )(rp, dv_is, a1, ab1.reshape(1, ah), a2)


def _shift_scores(scores, gmax):
    n = scores.shape[0]

    def body(s_ref, m_ref, o_ref):
        o_ref[...] = s_ref[...] - m_ref[0, 0]

    return pl.pallas_call(
        body,
        grid=(1,),
        in_specs=[
            pl.BlockSpec((n, 1), lambda i: (0, 0)),
            pl.BlockSpec((1, 1), lambda i: (0, 0)),
        ],
        out_specs=pl.BlockSpec((n, 1), lambda i: (0, 0)),
        out_shape=jax.ShapeDtypeStruct((n, 1), jnp.float32),
    )(scores, gmax)


def _rden_kernel(denp):
    def body(d_ref, o_ref):
        den = d_ref[0, :, 0:1] + d_ref[1, :, 0:1]
        o_ref[...] = 1.0 / jnp.maximum(den, 1e-12)

    return pl.pallas_call(
        body,
        grid=(1,),
        in_specs=[pl.BlockSpec((2, _E, _L), lambda i: (0, 0, 0))],
        out_specs=pl.BlockSpec((_E, 1), lambda i: (0, 0)),
        out_shape=jax.ShapeDtypeStruct((_E, 1), jnp.float32),
    )(denp)


def _decode(mu, zp, block_rows=1000):
    n, c = mu.shape

    def body(x_ref, z_ref, h_ref, z_out_ref):
        i = pl.program_id(0)
        z = z_ref[0] + z_ref[1]

        @pl.when(i == 0)
        def _():
            z_out_ref[...] = z
        acc = lax.dot_general(
            x_ref[...], z,
            dimension_numbers=(((1,), (1,)), ((), ())),
            preferred_element_type=jnp.float32)
        h_ref[...] = jax.nn.sigmoid(acc)

    return pl.pallas_call(
        body,
        grid=(n // block_rows,),
        in_specs=[
            pl.BlockSpec((block_rows, c), lambda i: (i, 0)),
            pl.BlockSpec((2, _E, c), lambda i: (0, 0, 0)),
        ],
        out_specs=[
            pl.BlockSpec((block_rows, _E), lambda i: (i, 0)),
            pl.BlockSpec((_E, c), lambda i: (0, 0)),
        ],
        out_shape=[
            jax.ShapeDtypeStruct((n, _E), jnp.float32),
            jax.ShapeDtypeStruct((_E, c), jnp.float32),
        ],
    )(mu, zp)


def kernel(X, vertex_idx, hyperedge_idx, W1, b1, W2, b2, W3, b3, A1, ab1, A2):
    c_out = W2.shape[1]
    pad = _NNZP - _NNZ

    vi = vertex_idx.astype(jnp.int32)
    ei = hyperedge_idx.astype(jnp.int32)
    v_g = jnp.concatenate([vi, jnp.zeros((pad,), jnp.int32)]).reshape(_ROWS, _CHA)
    v_s = jnp.concatenate([vi, jnp.full((pad,), _N, jnp.int32)]).reshape(_ROWS, _CHA)
    e_g = jnp.concatenate([ei, jnp.zeros((pad,), jnp.int32)]).reshape(_ROWS, _CHA)
    e_s = jnp.concatenate([ei, jnp.full((pad,), _E, jnp.int32)]).reshape(_ROWS, _CHA)

    ones_tab = jnp.ones((_CHA, _L), jnp.float32)
    z_n16 = jnp.zeros((_N + 1, _L), jnp.float32)
    z_e16 = jnp.zeros((_E + 1, _L), jnp.float32)
    z_e128 = jnp.zeros((_E + 1, 128), jnp.float32)
    z_e64 = jnp.zeros((_E + 1, 64), jnp.float32)
    z_n64 = jnp.zeros((_N + 1, 64), jnp.float32)
    z_row16 = jnp.zeros((_CHA, _L), jnp.float32)
    z_e32 = jnp.zeros((_E + 1, c_out), jnp.float32)

    dvp, dep = _sc_degrees(v_s, e_s, ones_tab, z_n16, z_e16)
    h0 = _mm_relu(X, W1, b1)
    h0s, dv_is, de_inv = _deg_combine_scale(dvp, dep, h0)

    yp = _sc_pair_pass(h0s, v_g, e_s, _E, z_e128)
    ys = _comb_scale(yp, de_inv, _E)
    ppa = _sc_pair_pass(ys[:, :64], e_g, v_s, _N, z_n64)
    ppb = _sc_pair_pass(ys[:, 64:], e_g, v_s, _N, z_n64)

    wc = jnp.concatenate([W2, W3], axis=1)
    bc = jnp.concatenate([b2, b3], axis=0)
    gs = _smooth_out_project(ppa, ppb, dv_is, wc, bc)
    qp = _sc_pair_pass(gs, v_g, e_s, _E, z_e64)
    y2s = _comb_scale(qp, de_inv, _E)
    rp = _sc_pair_pass(y2s, e_g, v_s, _N, z_n64)

    mu, logvar, scores, gmax = _final_smooth_attention(
        rp, dv_is, A1, ab1, A2, c_out)

    ssh = _shift_scores(scores, gmax).reshape(_N)
    ex, denp = _sc_att_ex(ssh, v_g, e_s, z_row16, z_e16)
    rden = _rden_kernel(denp).reshape(_E)
    beta, zpart = _sc_att_z(ex, rden, mu, v_g, e_g, e_s, z_e32)
    h_out, z = _decode(mu, zpart)

    return (mu, z, h_out, mu, logvar, beta[:_NNZ])


# speedup vs baseline: 8.8172x; 1.3609x over previous
import functools

import jax
import jax.numpy as jnp
from jax import lax
from jax.experimental import pallas as pl
from jax.experimental.pallas import tpu as pltpu
from jax.experimental.pallas import tpu_sc as plsc

_NC = 2
_NS = 16
_L = 16
_NW = _NC * _NS

_N = 10000
_E = 2000
_NNZ = 320000

_CHA = 64
_CHT = 160
_PT = _CHA * _CHT
_NNZP = _PT * _NW
_ROWS = _NNZP // _CHA
_R = 4


def _sc_mesh():
    return plsc.VectorSubcoreMesh(
        core_axis_name="c", subcore_axis_name="s",
        num_cores=_NC, num_subcores=_NS)


_SC_PARAMS = pltpu.CompilerParams(
    use_tc_tiling_on_sc=False, needs_layout_passes=False)


def _sc_degrees(v_s, e_s, ones_tab, zv, ze):
    @functools.partial(
        pl.kernel,
        out_type=(jax.ShapeDtypeStruct((_NC, _N + 1, _L), jnp.float32),
                  jax.ShapeDtypeStruct((_NC, _E + 1, _L), jnp.float32)),
        mesh=_sc_mesh(),
        compiler_params=_SC_PARAMS,
        scratch_types=[
            pltpu.VMEM((_CHT, _CHA), jnp.int32),
            pltpu.VMEM((_CHT, _CHA), jnp.int32),
            pltpu.VMEM((_CHA, _L), jnp.float32),
            pltpu.VMEM_SHARED((_N + 1, _L), jnp.float32),
            pltpu.VMEM_SHARED((_E + 1, _L), jnp.float32),
            pltpu.SemaphoreType.DMA,
            pltpu.SemaphoreType.DMA,
        ],
    )
    def kfn(v2_h, e2_h, ones_h, zv_h, ze_h, dvp_h, dep_h,
            vbuf, ebuf, ones_v, accv, acce, semv, seme):
        c = lax.axis_index("c")
        s = lax.axis_index("s")
        wid = c * _NS + s
        pltpu.sync_copy(v2_h.at[pl.ds(wid * _CHT, _CHT)], vbuf)
        pltpu.sync_copy(e2_h.at[pl.ds(wid * _CHT, _CHT)], ebuf)
        pltpu.sync_copy(ones_h, ones_v)

        @pl.when(s == 0)
        def _():
            pltpu.sync_copy(zv_h, accv)
            pltpu.sync_copy(ze_h, acce)
        plsc.subcore_barrier()

        @pl.loop(0, _CHT)
        def _(j):
            @pl.when(j >= _R)
            def _():
                pltpu.make_async_copy(
                    ones_v, accv.at[vbuf.at[j - _R]], semv).wait()
                pltpu.make_async_copy(
                    ones_v, acce.at[ebuf.at[j - _R]], seme).wait()
            pltpu.async_copy(ones_v, accv.at[vbuf.at[j]], semv, add=True)
            pltpu.async_copy(ones_v, acce.at[ebuf.at[j]], seme, add=True)

        for t in range(_R):
            pltpu.make_async_copy(
                ones_v, accv.at[vbuf.at[_CHT - _R + t]], semv).wait()
            pltpu.make_async_copy(
                ones_v, acce.at[ebuf.at[_CHT - _R + t]], seme).wait()

        plsc.subcore_barrier()

        @pl.when(s == 0)
        def _():
            pltpu.sync_copy(accv, dvp_h.at[c])
            pltpu.sync_copy(acce, dep_h.at[c])

    return kfn(v_s, e_s, ones_tab, zv, ze)


def _sc_pair_pass(table, g2, s2, out_rows, zeros_acc):
    _, w = table.shape

    @functools.partial(
        pl.kernel,
        out_type=jax.ShapeDtypeStruct((_NC, out_rows + 1, w), jnp.float32),
        mesh=_sc_mesh(),
        compiler_params=_SC_PARAMS,
        scratch_types=[
            pltpu.VMEM((_CHT, _CHA), jnp.int32),
            pltpu.VMEM((_CHT, _CHA), jnp.int32),
            [pltpu.VMEM((_CHA, w), jnp.float32) for _ in range(_R)],
            pltpu.VMEM_SHARED((out_rows + 1, w), jnp.float32),
            [pltpu.SemaphoreType.DMA for _ in range(_R)],
            [pltpu.SemaphoreType.DMA for _ in range(_R)],
        ],
    )
    def kfn(tab_h, g2_h, s2_h, zz_h, out_h,
            gbuf, sbuf, rows, acc, gsems, ssems):
        c = lax.axis_index("c")
        s = lax.axis_index("s")
        wid = c * _NS + s
        pltpu.sync_copy(g2_h.at[pl.ds(wid * _CHT, _CHT)], gbuf)
        pltpu.sync_copy(s2_h.at[pl.ds(wid * _CHT, _CHT)], sbuf)

        @pl.when(s == 0)
        def _():
            pltpu.sync_copy(zz_h, acc)
        plsc.subcore_barrier()

        pltpu.async_copy(tab_h.at[gbuf.at[0]], rows[0], gsems[0])
        pltpu.async_copy(tab_h.at[gbuf.at[1]], rows[1], gsems[1])

        @pl.loop(0, _CHT, step=_R)
        def _(j):
            for b in range(_R):
                k = j + b
                bp = (b + 2) % _R
                pltpu.make_async_copy(
                    tab_h.at[gbuf.at[k]], rows[b], gsems[b]).wait()
                pltpu.async_copy(
                    rows[b], acc.at[sbuf.at[k]], ssems[b], add=True)

                @pl.when(k >= 2)
                def _():
                    pltpu.make_async_copy(
                        rows[bp], acc.at[sbuf.at[k - 2]], ssems[bp]).wait()

                @pl.when(k + 2 < _CHT)
                def _():
                    pltpu.async_copy(
                        tab_h.at[gbuf.at[k + 2]], rows[bp], gsems[bp])

        for k in (_CHT - 2, _CHT - 1):
            pltpu.make_async_copy(
                rows[k % _R], acc.at[sbuf.at[k]], ssems[k % _R]).wait()

        plsc.subcore_barrier()

        @pl.when(s == 0)
        def _():
            pltpu.sync_copy(acc, out_h.at[c])

    return kfn(table, g2, s2, zeros_acc)


def _sc_att_ex(ssh, v_g, e_s, zrow, zden):
    @functools.partial(
        pl.kernel,
        out_type=(jax.ShapeDtypeStruct((_NNZP,), jnp.float32),
                  jax.ShapeDtypeStruct((_NC, _E + 1, _L), jnp.float32)),
        mesh=_sc_mesh(),
        compiler_params=_SC_PARAMS,
        scratch_types=[
            pltpu.VMEM((_CHT, _CHA), jnp.int32),
            pltpu.VMEM((_CHT, _CHA), jnp.int32),
            pltpu.VMEM((_N,), jnp.float32),
            pltpu.VMEM((_PT,), jnp.float32),
            [pltpu.VMEM((_CHA, _L), jnp.float32) for _ in range(_R)],
            pltpu.VMEM_SHARED((_E + 1, _L), jnp.float32),
            [pltpu.SemaphoreType.DMA for _ in range(_R)],
        ],
    )
    def kfn(ssh_h, v2_h, e2_h, zrow_h, zden_h, ex_h, denp_h,
            vbuf, ebuf, stab, extile, exbufs, accd, ssems):
        c = lax.axis_index("c")
        s = lax.axis_index("s")
        wid = c * _NS + s
        pltpu.sync_copy(v2_h.at[pl.ds(wid * _CHT, _CHT)], vbuf)
        pltpu.sync_copy(e2_h.at[pl.ds(wid * _CHT, _CHT)], ebuf)
        pltpu.sync_copy(ssh_h, stab)
        for b in range(_R):
            pltpu.sync_copy(zrow_h, exbufs[b])

        @pl.when(s == 0)
        def _():
            pltpu.sync_copy(zden_h, accd)
        plsc.subcore_barrier()

        zeros16 = jnp.zeros((_L,), jnp.int32)

        @pl.loop(0, _CHT, step=_R)
        def _(j):
            for b in range(_R):
                k = j + b

                @pl.when(k >= _R)
                def _():
                    pltpu.make_async_copy(
                        exbufs[b], accd.at[ebuf.at[k - _R]], ssems[b]).wait()
                for u in range(_CHA // _L):
                    idx = vbuf[k, pl.ds(u * _L, _L)]
                    ex = jnp.exp(plsc.load_gather(stab, [idx]))
                    extile[pl.ds(k * _CHA + u * _L, _L)] = ex
                    rowi = lax.iota(jnp.int32, _L) + (u * _L)
                    plsc.store_scatter(exbufs[b], [rowi, zeros16], ex)
                pltpu.async_copy(
                    exbufs[b], accd.at[ebuf.at[k]], ssems[b], add=True)

        for b in range(_R):
            pltpu.make_async_copy(
                exbufs[b], accd.at[ebuf.at[_CHT - _R + b]], ssems[b]).wait()

        plsc.subcore_barrier()
        pltpu.sync_copy(extile, ex_h.at[pl.ds(wid * _PT, _PT)])

        @pl.when(s == 0)
        def _():
            pltpu.sync_copy(accd, denp_h.at[c])

    return kfn(ssh, v_g, e_s, zrow, zden)


def _sc_att_z(ex, rden, mu, v_g, e_g, e_s, zz):
    c_out = mu.shape[1]

    @functools.partial(
        pl.kernel,
        out_type=(jax.ShapeDtypeStruct((_NNZP,), jnp.float32),
                  jax.ShapeDtypeStruct((_NC, _E + 1, c_out), jnp.float32)),
        mesh=_sc_mesh(),
        compiler_params=_SC_PARAMS,
        scratch_types=[
            pltpu.VMEM((_CHT, _CHA), jnp.int32),
            pltpu.VMEM((_CHT, _CHA), jnp.int32),
            pltpu.VMEM((_CHT, _CHA), jnp.int32),
            pltpu.VMEM((_E,), jnp.float32),
            pltpu.VMEM((_PT,), jnp.float32),
            pltpu.VMEM((_PT,), jnp.float32),
            [pltpu.VMEM((_CHA, c_out), jnp.float32) for _ in range(_R)],
            pltpu.VMEM_SHARED((_E + 1, c_out), jnp.float32),
            [pltpu.SemaphoreType.DMA for _ in range(_R)],
            [pltpu.SemaphoreType.DMA for _ in range(_R)],
        ],
    )
    def kfn(ex_h, rden_h, mu_h, vg_h, eg_h, es_h, zz_h, beta_h, zp_h,
            vbuf, egbuf, esbuf, rdtab, extile, betatile, rows,
            accz, gsems, ssems):
        c = lax.axis_index("c")
        s = lax.axis_index("s")
        wid = c * _NS + s
        pltpu.sync_copy(vg_h.at[pl.ds(wid * _CHT, _CHT)], vbuf)
        pltpu.sync_copy(eg_h.at[pl.ds(wid * _CHT, _CHT)], egbuf)
        pltpu.sync_copy(es_h.at[pl.ds(wid * _CHT, _CHT)], esbuf)
        pltpu.sync_copy(rden_h, rdtab)
        pltpu.sync_copy(ex_h.at[pl.ds(wid * _PT, _PT)], extile)

        @pl.when(s == 0)
        def _():
            pltpu.sync_copy(zz_h, accz)
        plsc.subcore_barrier()

        pltpu.async_copy(mu_h.at[vbuf.at[0]], rows[0], gsems[0])
        pltpu.async_copy(mu_h.at[vbuf.at[1]], rows[1], gsems[1])

        @pl.loop(0, _CHT, step=_R)
        def _(j):
            for b in range(_R):
                k = j + b
                bp = (b + 2) % _R
                pltpu.make_async_copy(
                    mu_h.at[vbuf.at[k]], rows[b], gsems[b]).wait()
                for u in range(_CHA // _L):
                    eidx = egbuf[k, pl.ds(u * _L, _L)]
                    rd = plsc.load_gather(rdtab, [eidx])
                    exv = extile[pl.ds(k * _CHA + u * _L, _L)]
                    bet = exv * rd
                    betatile[pl.ds(k * _CHA + u * _L, _L)] = bet
                    rowi = lax.iota(jnp.int32, _L) + (u * _L)
                    for col in range(c_out):
                        ci = jnp.full((_L,), col, jnp.int32)
                        vals = plsc.load_gather(rows[b], [rowi, ci])
                        plsc.store_scatter(rows[b], [rowi, ci], vals * bet)
                pltpu.async_copy(
                    rows[b], accz.at[esbuf.at[k]], ssems[b], add=True)

                @pl.when(k >= 2)
                def _():
                    pltpu.make_async_copy(
                        rows[bp], accz.at[esbuf.at[k - 2]], ssems[bp]).wait()

                @pl.when(k + 2 < _CHT)
                def _():
                    pltpu.async_copy(
                        mu_h.at[vbuf.at[k + 2]], rows[bp], gsems[bp])

        for k in (_CHT - 2, _CHT - 1):
            pltpu.make_async_copy(
                rows[k % _R], accz.at[esbuf.at[k]], ssems[k % _R]).wait()

        plsc.subcore_barrier()
        pltpu.sync_copy(betatile, beta_h.at[pl.ds(wid * _PT, _PT)])

        @pl.when(s == 0)
        def _():
            pltpu.sync_copy(accz, zp_h.at[c])

    return kfn(ex, rden, mu, v_g, e_g, e_s, zz)


def _mm_relu(x, w, b, block_rows=1000):
    n, k = x.shape
    _, m = w.shape

    def body(x_ref, w_ref, b_ref, o_ref):
        acc = jnp.dot(x_ref[...], w_ref[...], preferred_element_type=jnp.float32)
        o_ref[...] = jax.nn.relu(acc + b_ref[...])

    return pl.pallas_call(
        body,
        grid=(n // block_rows,),
        in_specs=[
            pl.BlockSpec((block_rows, k), lambda i: (i, 0)),
            pl.BlockSpec((k, m), lambda i: (0, 0)),
            pl.BlockSpec((1, m), lambda i: (0, 0)),
        ],
        out_specs=pl.BlockSpec((block_rows, m), lambda i: (i, 0)),
        out_shape=jax.ShapeDtypeStruct((n, m), jnp.float32),
    )(x, w, b.reshape(1, m))


def _deg_project(dvp, dep, h0, wc, block_rows=1000):
    n = h0.shape[0]
    cin = h0.shape[1]
    m = wc.shape[1]

    def body(dv_ref, de_ref, h_ref, w_ref, x0_ref, dvis_ref, dei_ref):
        i = pl.program_id(0)
        dv = dv_ref[0, :, 0:1] + dv_ref[1, :, 0:1]
        dvis = jnp.where(dv > 0, lax.rsqrt(jnp.maximum(dv, 1e-12)), 0.0)
        dvis_ref[...] = dvis
        x0_ref[...] = jnp.dot(h_ref[...] * dvis, w_ref[...],
                              preferred_element_type=jnp.float32)

        @pl.when(i == 0)
        def _():
            de = de_ref[0, :, 0:1] + de_ref[1, :, 0:1]
            dei_ref[...] = jnp.where(de > 0, 1.0 / jnp.maximum(de, 1e-12), 0.0)

    return pl.pallas_call(
        body,
        grid=(n // block_rows,),
        in_specs=[
            pl.BlockSpec((2, block_rows, _L), lambda i: (0, i, 0)),
            pl.BlockSpec((2, _E, _L), lambda i: (0, 0, 0)),
            pl.BlockSpec((block_rows, cin), lambda i: (i, 0)),
            pl.BlockSpec((cin, m), lambda i: (0, 0)),
        ],
        out_specs=[
            pl.BlockSpec((block_rows, m), lambda i: (i, 0)),
            pl.BlockSpec((block_rows, 1), lambda i: (i, 0)),
            pl.BlockSpec((_E, 1), lambda i: (0, 0)),
        ],
        out_shape=[
            jax.ShapeDtypeStruct((n, m), jnp.float32),
            jax.ShapeDtypeStruct((n, 1), jnp.float32),
            jax.ShapeDtypeStruct((_E, 1), jnp.float32),
        ],
    )(dvp, dep, h0, wc)


def _comb_scale(yp, scale, rows):
    _, _, w = yp.shape

    def body(y_ref, s_ref, o_ref):
        o_ref[...] = (y_ref[0] + y_ref[1]) * s_ref[...]

    return pl.pallas_call(
        body,
        grid=(1,),
        in_specs=[
            pl.BlockSpec((2, rows, w), lambda i: (0, 0, 0)),
            pl.BlockSpec((rows, 1), lambda i: (0, 0)),
        ],
        out_specs=pl.BlockSpec((rows, w), lambda i: (0, 0)),
        out_shape=jax.ShapeDtypeStruct((rows, w), jnp.float32),
    )(yp, scale)


def _mid_project(pp, dv_is, bc, block_rows=1000):
    _, _, w = pp.shape
    n = dv_is.shape[0]

    def body(p_ref, d_ref, b_ref, o_ref):
        d = d_ref[...]
        o_ref[...] = d * d * (p_ref[0] + p_ref[1]) + d * b_ref[...]

    return pl.pallas_call(
        body,
        grid=(n // block_rows,),
        in_specs=[
            pl.BlockSpec((2, block_rows, w), lambda i: (0, i, 0)),
            pl.BlockSpec((block_rows, 1), lambda i: (i, 0)),
            pl.BlockSpec((1, w), lambda i: (0, 0)),
        ],
        out_specs=pl.BlockSpec((block_rows, w), lambda i: (i, 0)),
        out_shape=jax.ShapeDtypeStruct((n, w), jnp.float32),
    )(pp, dv_is, bc.reshape(1, w))


def _final_smooth_attention(rp, dv_is, a1, ab1, a2, c_out, block_rows=1000):
    _, _, w = rp.shape
    n = dv_is.shape[0]
    ah = a1.shape[1]

    def body(r_ref, d_ref, a1_ref, ab1_ref, a2_ref,
             mu_ref, lv_ref, sc_ref, mx_ref):
        i = pl.program_id(0)
        s2 = (r_ref[0] + r_ref[1]) * d_ref[...]
        mu = s2[:, :c_out]
        mu_ref[...] = mu
        lv_ref[...] = s2[:, c_out:]
        t = jnp.tanh(jnp.dot(mu, a1_ref[...], preferred_element_type=jnp.float32)
                     + ab1_ref[...])
        sc = jnp.dot(t, a2_ref[...], preferred_element_type=jnp.float32)
        sc_ref[...] = sc

        @pl.when(i == 0)
        def _():
            mx_ref[...] = jnp.full_like(mx_ref[...], -jnp.inf)
        mx_ref[...] = jnp.maximum(mx_ref[...], jnp.max(sc))

    return pl.pallas_call(
        body,
        grid=(n // block_rows,),
        in_specs=[
            pl.BlockSpec((2, block_rows, w), lambda i: (0, i, 0)),
            pl.BlockSpec((block_rows, 1), lambda i: (i, 0)),
            pl.BlockSpec((c_out, ah), lambda i: (0, 0)),
            pl.BlockSpec((1, ah), lambda i: (0, 0)),
            pl.BlockSpec((ah, 1), lambda i: (0, 0)),
        ],
        out_specs=[
            pl.BlockSpec((block_rows, c_out), lambda i: (i, 0)),
            pl.BlockSpec((block_rows, c_out), lambda i: (i, 0)),
            pl.BlockSpec((block_rows, 1), lambda i: (i, 0)),
            pl.BlockSpec((1, 1), lambda i: (0, 0)),
        ],
        out_shape=[
            jax.ShapeDtypeStruct((n, c_out), jnp.float32),
            jax.ShapeDtypeStruct((n, c_out), jnp.float32),
            jax.ShapeDtypeStruct((n, 1), jnp.float32),
            jax.ShapeDtypeStruct((1, 1), jnp.float32),
        ],
    )(rp, dv_is, a1, ab1.reshape(1, ah), a2)


def _shift_scores(scores, gmax):
    n = scores.shape[0]

    def body(s_ref, m_ref, o_ref):
        o_ref[...] = s_ref[...] - m_ref[0, 0]

    return pl.pallas_call(
        body,
        grid=(1,),
        in_specs=[
            pl.BlockSpec((n, 1), lambda i: (0, 0)),
            pl.BlockSpec((1, 1), lambda i: (0, 0)),
        ],
        out_specs=pl.BlockSpec((n, 1), lambda i: (0, 0)),
        out_shape=jax.ShapeDtypeStruct((n, 1), jnp.float32),
    )(scores, gmax)


def _rden_kernel(denp):
    def body(d_ref, o_ref):
        den = d_ref[0, :, 0:1] + d_ref[1, :, 0:1]
        o_ref[...] = 1.0 / jnp.maximum(den, 1e-12)

    return pl.pallas_call(
        body,
        grid=(1,),
        in_specs=[pl.BlockSpec((2, _E, _L), lambda i: (0, 0, 0))],
        out_specs=pl.BlockSpec((_E, 1), lambda i: (0, 0)),
        out_shape=jax.ShapeDtypeStruct((_E, 1), jnp.float32),
    )(denp)


def _decode(mu, zp, block_rows=1000):
    n, c = mu.shape

    def body(x_ref, z_ref, h_ref, z_out_ref):
        i = pl.program_id(0)
        z = z_ref[0] + z_ref[1]

        @pl.when(i == 0)
        def _():
            z_out_ref[...] = z
        acc = lax.dot_general(
            x_ref[...], z,
            dimension_numbers=(((1,), (1,)), ((), ())),
            preferred_element_type=jnp.float32)
        h_ref[...] = jax.nn.sigmoid(acc)

    return pl.pallas_call(
        body,
        grid=(n // block_rows,),
        in_specs=[
            pl.BlockSpec((block_rows, c), lambda i: (i, 0)),
            pl.BlockSpec((2, _E, c), lambda i: (0, 0, 0)),
        ],
        out_specs=[
            pl.BlockSpec((block_rows, _E), lambda i: (i, 0)),
            pl.BlockSpec((_E, c), lambda i: (0, 0)),
        ],
        out_shape=[
            jax.ShapeDtypeStruct((n, _E), jnp.float32),
            jax.ShapeDtypeStruct((_E, c), jnp.float32),
        ],
    )(mu, zp)


def kernel(X, vertex_idx, hyperedge_idx, W1, b1, W2, b2, W3, b3, A1, ab1, A2):
    c_out = W2.shape[1]
    pad = _NNZP - _NNZ

    vi = vertex_idx.astype(jnp.int32)
    ei = hyperedge_idx.astype(jnp.int32)
    v_g = jnp.concatenate([vi, jnp.zeros((pad,), jnp.int32)]).reshape(_ROWS, _CHA)
    v_s = jnp.concatenate([vi, jnp.full((pad,), _N, jnp.int32)]).reshape(_ROWS, _CHA)
    e_g = jnp.concatenate([ei, jnp.zeros((pad,), jnp.int32)]).reshape(_ROWS, _CHA)
    e_s = jnp.concatenate([ei, jnp.full((pad,), _E, jnp.int32)]).reshape(_ROWS, _CHA)

    ones_tab = jnp.ones((_CHA, _L), jnp.float32)
    z_n16 = jnp.zeros((_N + 1, _L), jnp.float32)
    z_e16 = jnp.zeros((_E + 1, _L), jnp.float32)
    z_e64 = jnp.zeros((_E + 1, 64), jnp.float32)
    z_n64 = jnp.zeros((_N + 1, 64), jnp.float32)
    z_row16 = jnp.zeros((_CHA, _L), jnp.float32)
    z_e32 = jnp.zeros((_E + 1, c_out), jnp.float32)

    dvp, dep = _sc_degrees(v_s, e_s, ones_tab, z_n16, z_e16)
    h0 = _mm_relu(X, W1, b1)

    wc = jnp.concatenate([W2, W3], axis=1)
    bc = jnp.concatenate([b2, b3], axis=0)
    x0, dv_is, de_inv = _deg_project(dvp, dep, h0, wc)

    yp = _sc_pair_pass(x0, v_g, e_s, _E, z_e64)
    ys = _comb_scale(yp, de_inv, _E)
    pp = _sc_pair_pass(ys, e_g, v_s, _N, z_n64)
    gs = _mid_project(pp, dv_is, bc)

    qp = _sc_pair_pass(gs, v_g, e_s, _E, z_e64)
    y2s = _comb_scale(qp, de_inv, _E)
    rp = _sc_pair_pass(y2s, e_g, v_s, _N, z_n64)

    mu, logvar, scores, gmax = _final_smooth_attention(
        rp, dv_is, A1, ab1, A2, c_out)

    ssh = _shift_scores(scores, gmax).reshape(_N)
    ex, denp = _sc_att_ex(ssh, v_g, e_s, z_row16, z_e16)
    rden = _rden_kernel(denp).reshape(_E)
    beta, zpart = _sc_att_z(ex, rden, mu, v_g, e_g, e_s, z_e32)
    h_out, z = _decode(mu, zpart)

    return (mu, z, h_out, mu, logvar, beta[:_NNZ])


# speedup vs baseline: 14.6446x; 1.6609x over previous
import functools

import jax
import jax.numpy as jnp
from jax import lax
from jax.experimental import pallas as pl
from jax.experimental.pallas import tpu as pltpu
from jax.experimental.pallas import tpu_sc as plsc

_NC = 2
_NS = 16
_L = 16
_NW = _NC * _NS

_N = 10000
_E = 2000
_NNZ = 320000

_CHA = 64
_CHT = 160
_PT = _CHA * _CHT
_NNZP = _PT * _NW
_ROWS = _NNZP // _CHA
_R = 4


def _sc_mesh():
    return plsc.VectorSubcoreMesh(
        core_axis_name="c", subcore_axis_name="s",
        num_cores=_NC, num_subcores=_NS)


_SC_PARAMS = pltpu.CompilerParams(
    use_tc_tiling_on_sc=False, needs_layout_passes=False)


def _sc_degrees(v_s, e_s, ones_tab, zv, ze):
    @functools.partial(
        pl.kernel,
        out_type=(jax.ShapeDtypeStruct((_NC, _N + 1, _L), jnp.float32),
                  jax.ShapeDtypeStruct((_NC, _E + 1, _L), jnp.float32)),
        mesh=_sc_mesh(),
        compiler_params=_SC_PARAMS,
        scratch_types=[
            pltpu.VMEM((_CHT, _CHA), jnp.int32),
            pltpu.VMEM((_CHT, _CHA), jnp.int32),
            pltpu.VMEM((_CHA, _L), jnp.float32),
            pltpu.VMEM_SHARED((_N + 1, _L), jnp.float32),
            pltpu.VMEM_SHARED((_E + 1, _L), jnp.float32),
            pltpu.SemaphoreType.DMA,
            pltpu.SemaphoreType.DMA,
        ],
    )
    def kfn(v2_h, e2_h, ones_h, zv_h, ze_h, dvp_h, dep_h,
            vbuf, ebuf, ones_v, accv, acce, semv, seme):
        c = lax.axis_index("c")
        s = lax.axis_index("s")
        wid = c * _NS + s
        pltpu.sync_copy(v2_h.at[pl.ds(wid * _CHT, _CHT)], vbuf)
        pltpu.sync_copy(e2_h.at[pl.ds(wid * _CHT, _CHT)], ebuf)
        pltpu.sync_copy(ones_h, ones_v)

        @pl.when(s == 0)
        def _():
            pltpu.sync_copy(zv_h, accv)
            pltpu.sync_copy(ze_h, acce)
        plsc.subcore_barrier()

        @pl.loop(0, _CHT)
        def _(j):
            @pl.when(j >= _R)
            def _():
                pltpu.make_async_copy(
                    ones_v, accv.at[vbuf.at[j - _R]], semv).wait()
                pltpu.make_async_copy(
                    ones_v, acce.at[ebuf.at[j - _R]], seme).wait()
            pltpu.async_copy(ones_v, accv.at[vbuf.at[j]], semv, add=True)
            pltpu.async_copy(ones_v, acce.at[ebuf.at[j]], seme, add=True)

        for t in range(_R):
            pltpu.make_async_copy(
                ones_v, accv.at[vbuf.at[_CHT - _R + t]], semv).wait()
            pltpu.make_async_copy(
                ones_v, acce.at[ebuf.at[_CHT - _R + t]], seme).wait()

        plsc.subcore_barrier()

        @pl.when(s == 0)
        def _():
            pltpu.sync_copy(accv, dvp_h.at[c])
            pltpu.sync_copy(acce, dep_h.at[c])

    return kfn(v_s, e_s, ones_tab, zv, ze)


def _sc_pair_pass(table, g2, s2, out_rows, zeros_acc):
    rt, w = table.shape

    @functools.partial(
        pl.kernel,
        out_type=jax.ShapeDtypeStruct((_NC, out_rows + 1, w), jnp.float32),
        mesh=_sc_mesh(),
        compiler_params=_SC_PARAMS,
        scratch_types=[
            pltpu.VMEM((_CHT, _CHA), jnp.int32),
            pltpu.VMEM((_CHT, _CHA), jnp.int32),
            [pltpu.VMEM((_CHA, w), jnp.float32) for _ in range(_R)],
            pltpu.VMEM_SHARED((rt, w), jnp.float32),
            pltpu.VMEM_SHARED((out_rows + 1, w), jnp.float32),
            [pltpu.SemaphoreType.DMA for _ in range(_R)],
            [pltpu.SemaphoreType.DMA for _ in range(_R)],
        ],
    )
    def kfn(tab_h, g2_h, s2_h, zz_h, out_h,
            gbuf, sbuf, rows, tab_sh, acc, gsems, ssems):
        c = lax.axis_index("c")
        s = lax.axis_index("s")
        wid = c * _NS + s
        pltpu.sync_copy(g2_h.at[pl.ds(wid * _CHT, _CHT)], gbuf)
        pltpu.sync_copy(s2_h.at[pl.ds(wid * _CHT, _CHT)], sbuf)

        @pl.when(s == 0)
        def _():
            pltpu.sync_copy(zz_h, acc)

        @pl.when(s == 1)
        def _():
            pltpu.sync_copy(tab_h, tab_sh)
        plsc.subcore_barrier()

        pltpu.async_copy(tab_sh.at[gbuf.at[0]], rows[0], gsems[0])
        pltpu.async_copy(tab_sh.at[gbuf.at[1]], rows[1], gsems[1])

        @pl.loop(0, _CHT, step=_R)
        def _(j):
            for b in range(_R):
                k = j + b
                bp = (b + 2) % _R
                pltpu.make_async_copy(
                    tab_sh.at[gbuf.at[k]], rows[b], gsems[b]).wait()
                pltpu.async_copy(
                    rows[b], acc.at[sbuf.at[k]], ssems[b], add=True)

                @pl.when(k >= 2)
                def _():
                    pltpu.make_async_copy(
                        rows[bp], acc.at[sbuf.at[k - 2]], ssems[bp]).wait()

                @pl.when(k + 2 < _CHT)
                def _():
                    pltpu.async_copy(
                        tab_sh.at[gbuf.at[k + 2]], rows[bp], gsems[bp])

        for k in (_CHT - 2, _CHT - 1):
            pltpu.make_async_copy(
                rows[k % _R], acc.at[sbuf.at[k]], ssems[k % _R]).wait()

        plsc.subcore_barrier()

        @pl.when(s == 0)
        def _():
            pltpu.sync_copy(acc, out_h.at[c])

    return kfn(table, g2, s2, zeros_acc)


def _sc_att_ex(ssh, v_g, e_s, zrow, zden):
    @functools.partial(
        pl.kernel,
        out_type=(jax.ShapeDtypeStruct((_NNZP,), jnp.float32),
                  jax.ShapeDtypeStruct((_NC, _E + 1, _L), jnp.float32)),
        mesh=_sc_mesh(),
        compiler_params=_SC_PARAMS,
        scratch_types=[
            pltpu.VMEM((_CHT, _CHA), jnp.int32),
            pltpu.VMEM((_CHT, _CHA), jnp.int32),
            pltpu.VMEM((_N,), jnp.float32),
            pltpu.VMEM((_PT,), jnp.float32),
            [pltpu.VMEM((_CHA, _L), jnp.float32) for _ in range(_R)],
            pltpu.VMEM_SHARED((_E + 1, _L), jnp.float32),
            [pltpu.SemaphoreType.DMA for _ in range(_R)],
        ],
    )
    def kfn(ssh_h, v2_h, e2_h, zrow_h, zden_h, ex_h, denp_h,
            vbuf, ebuf, stab, extile, exbufs, accd, ssems):
        c = lax.axis_index("c")
        s = lax.axis_index("s")
        wid = c * _NS + s
        pltpu.sync_copy(v2_h.at[pl.ds(wid * _CHT, _CHT)], vbuf)
        pltpu.sync_copy(e2_h.at[pl.ds(wid * _CHT, _CHT)], ebuf)
        pltpu.sync_copy(ssh_h, stab)
        for b in range(_R):
            pltpu.sync_copy(zrow_h, exbufs[b])

        @pl.when(s == 0)
        def _():
            pltpu.sync_copy(zden_h, accd)
        plsc.subcore_barrier()

        zeros16 = jnp.zeros((_L,), jnp.int32)

        @pl.loop(0, _CHT, step=_R)
        def _(j):
            for b in range(_R):
                k = j + b

                @pl.when(k >= _R)
                def _():
                    pltpu.make_async_copy(
                        exbufs[b], accd.at[ebuf.at[k - _R]], ssems[b]).wait()
                for u in range(_CHA // _L):
                    idx = vbuf[k, pl.ds(u * _L, _L)]
                    ex = jnp.exp(plsc.load_gather(stab, [idx]))
                    extile[pl.ds(k * _CHA + u * _L, _L)] = ex
                    rowi = lax.iota(jnp.int32, _L) + (u * _L)
                    plsc.store_scatter(exbufs[b], [rowi, zeros16], ex)
                pltpu.async_copy(
                    exbufs[b], accd.at[ebuf.at[k]], ssems[b], add=True)

        for b in range(_R):
            pltpu.make_async_copy(
                exbufs[b], accd.at[ebuf.at[_CHT - _R + b]], ssems[b]).wait()

        plsc.subcore_barrier()
        pltpu.sync_copy(extile, ex_h.at[pl.ds(wid * _PT, _PT)])

        @pl.when(s == 0)
        def _():
            pltpu.sync_copy(accd, denp_h.at[c])

    return kfn(ssh, v_g, e_s, zrow, zden)


def _sc_att_z(ex, rden, mu, v_g, e_g, e_s, zz):
    c_out = mu.shape[1]

    @functools.partial(
        pl.kernel,
        out_type=(jax.ShapeDtypeStruct((_NNZP,), jnp.float32),
                  jax.ShapeDtypeStruct((_NC, _E + 1, c_out), jnp.float32)),
        mesh=_sc_mesh(),
        compiler_params=_SC_PARAMS,
        scratch_types=[
            pltpu.VMEM((_CHT, _CHA), jnp.int32),
            pltpu.VMEM((_CHT, _CHA), jnp.int32),
            pltpu.VMEM((_CHT, _CHA), jnp.int32),
            pltpu.VMEM((_E,), jnp.float32),
            pltpu.VMEM((_PT,), jnp.float32),
            pltpu.VMEM((_PT,), jnp.float32),
            [pltpu.VMEM((_CHA, c_out), jnp.float32) for _ in range(_R)],
            pltpu.VMEM_SHARED((_N, c_out), jnp.float32),
            pltpu.VMEM_SHARED((_E + 1, c_out), jnp.float32),
            [pltpu.SemaphoreType.DMA for _ in range(_R)],
            [pltpu.SemaphoreType.DMA for _ in range(_R)],
        ],
    )
    def kfn(ex_h, rden_h, mu_h, vg_h, eg_h, es_h, zz_h, beta_h, zp_h,
            vbuf, egbuf, esbuf, rdtab, extile, betatile, rows,
            mu_sh, accz, gsems, ssems):
        c = lax.axis_index("c")
        s = lax.axis_index("s")
        wid = c * _NS + s
        pltpu.sync_copy(vg_h.at[pl.ds(wid * _CHT, _CHT)], vbuf)
        pltpu.sync_copy(eg_h.at[pl.ds(wid * _CHT, _CHT)], egbuf)
        pltpu.sync_copy(es_h.at[pl.ds(wid * _CHT, _CHT)], esbuf)
        pltpu.sync_copy(rden_h, rdtab)
        pltpu.sync_copy(ex_h.at[pl.ds(wid * _PT, _PT)], extile)

        @pl.when(s == 0)
        def _():
            pltpu.sync_copy(zz_h, accz)

        @pl.when(s == 1)
        def _():
            pltpu.sync_copy(mu_h, mu_sh)
        plsc.subcore_barrier()

        pltpu.async_copy(mu_sh.at[vbuf.at[0]], rows[0], gsems[0])
        pltpu.async_copy(mu_sh.at[vbuf.at[1]], rows[1], gsems[1])

        @pl.loop(0, _CHT, step=_R)
        def _(j):
            for b in range(_R):
                k = j + b
                bp = (b + 2) % _R
                pltpu.make_async_copy(
                    mu_sh.at[vbuf.at[k]], rows[b], gsems[b]).wait()
                for u in range(_CHA // _L):
                    eidx = egbuf[k, pl.ds(u * _L, _L)]
                    rd = plsc.load_gather(rdtab, [eidx])
                    exv = extile[pl.ds(k * _CHA + u * _L, _L)]
                    bet = exv * rd
                    betatile[pl.ds(k * _CHA + u * _L, _L)] = bet
                    rowi = lax.iota(jnp.int32, _L) + (u * _L)
                    for col in range(c_out):
                        ci = jnp.full((_L,), col, jnp.int32)
                        vals = plsc.load_gather(rows[b], [rowi, ci])
                        plsc.store_scatter(rows[b], [rowi, ci], vals * bet)
                pltpu.async_copy(
                    rows[b], accz.at[esbuf.at[k]], ssems[b], add=True)

                @pl.when(k >= 2)
                def _():
                    pltpu.make_async_copy(
                        rows[bp], accz.at[esbuf.at[k - 2]], ssems[bp]).wait()

                @pl.when(k + 2 < _CHT)
                def _():
                    pltpu.async_copy(
                        mu_sh.at[vbuf.at[k + 2]], rows[bp], gsems[bp])

        for k in (_CHT - 2, _CHT - 1):
            pltpu.make_async_copy(
                rows[k % _R], accz.at[esbuf.at[k]], ssems[k % _R]).wait()

        plsc.subcore_barrier()
        pltpu.sync_copy(betatile, beta_h.at[pl.ds(wid * _PT, _PT)])

        @pl.when(s == 0)
        def _():
            pltpu.sync_copy(accz, zp_h.at[c])

    return kfn(ex, rden, mu, v_g, e_g, e_s, zz)


def _mm_relu(x, w, b, block_rows=1000):
    n, k = x.shape
    _, m = w.shape

    def body(x_ref, w_ref, b_ref, o_ref):
        acc = jnp.dot(x_ref[...], w_ref[...], preferred_element_type=jnp.float32)
        o_ref[...] = jax.nn.relu(acc + b_ref[...])

    return pl.pallas_call(
        body,
        grid=(n // block_rows,),
        in_specs=[
            pl.BlockSpec((block_rows, k), lambda i: (i, 0)),
            pl.BlockSpec((k, m), lambda i: (0, 0)),
            pl.BlockSpec((1, m), lambda i: (0, 0)),
        ],
        out_specs=pl.BlockSpec((block_rows, m), lambda i: (i, 0)),
        out_shape=jax.ShapeDtypeStruct((n, m), jnp.float32),
    )(x, w, b.reshape(1, m))


def _deg_project(dvp, dep, h0, wc, block_rows=1000):
    n = h0.shape[0]
    cin = h0.shape[1]
    m = wc.shape[1]

    def body(dv_ref, de_ref, h_ref, w_ref, x0_ref, dvis_ref, dei_ref):
        i = pl.program_id(0)
        dv = dv_ref[0, :, 0:1] + dv_ref[1, :, 0:1]
        dvis = jnp.where(dv > 0, lax.rsqrt(jnp.maximum(dv, 1e-12)), 0.0)
        dvis_ref[...] = dvis
        x0_ref[...] = jnp.dot(h_ref[...] * dvis, w_ref[...],
                              preferred_element_type=jnp.float32)

        @pl.when(i == 0)
        def _():
            de = de_ref[0, :, 0:1] + de_ref[1, :, 0:1]
            dei_ref[...] = jnp.where(de > 0, 1.0 / jnp.maximum(de, 1e-12), 0.0)

    return pl.pallas_call(
        body,
        grid=(n // block_rows,),
        in_specs=[
            pl.BlockSpec((2, block_rows, _L), lambda i: (0, i, 0)),
            pl.BlockSpec((2, _E, _L), lambda i: (0, 0, 0)),
            pl.BlockSpec((block_rows, cin), lambda i: (i, 0)),
            pl.BlockSpec((cin, m), lambda i: (0, 0)),
        ],
        out_specs=[
            pl.BlockSpec((block_rows, m), lambda i: (i, 0)),
            pl.BlockSpec((block_rows, 1), lambda i: (i, 0)),
            pl.BlockSpec((_E, 1), lambda i: (0, 0)),
        ],
        out_shape=[
            jax.ShapeDtypeStruct((n, m), jnp.float32),
            jax.ShapeDtypeStruct((n, 1), jnp.float32),
            jax.ShapeDtypeStruct((_E, 1), jnp.float32),
        ],
    )(dvp, dep, h0, wc)


def _comb_scale(yp, scale, rows):
    _, _, w = yp.shape

    def body(y_ref, s_ref, o_ref):
        o_ref[...] = (y_ref[0] + y_ref[1]) * s_ref[...]

    return pl.pallas_call(
        body,
        grid=(1,),
        in_specs=[
            pl.BlockSpec((2, rows, w), lambda i: (0, 0, 0)),
            pl.BlockSpec((rows, 1), lambda i: (0, 0)),
        ],
        out_specs=pl.BlockSpec((rows, w), lambda i: (0, 0)),
        out_shape=jax.ShapeDtypeStruct((rows, w), jnp.float32),
    )(yp, scale)


def _mid_project(pp, dv_is, bc, block_rows=1000):
    _, _, w = pp.shape
    n = dv_is.shape[0]

    def body(p_ref, d_ref, b_ref, o_ref):
        d = d_ref[...]
        o_ref[...] = d * d * (p_ref[0] + p_ref[1]) + d * b_ref[...]

    return pl.pallas_call(
        body,
        grid=(n // block_rows,),
        in_specs=[
            pl.BlockSpec((2, block_rows, w), lambda i: (0, i, 0)),
            pl.BlockSpec((block_rows, 1), lambda i: (i, 0)),
            pl.BlockSpec((1, w), lambda i: (0, 0)),
        ],
        out_specs=pl.BlockSpec((block_rows, w), lambda i: (i, 0)),
        out_shape=jax.ShapeDtypeStruct((n, w), jnp.float32),
    )(pp, dv_is, bc.reshape(1, w))


def _final_smooth_attention(rp, dv_is, a1, ab1, a2, c_out, block_rows=1000):
    _, _, w = rp.shape
    n = dv_is.shape[0]
    ah = a1.shape[1]

    def body(r_ref, d_ref, a1_ref, ab1_ref, a2_ref,
             mu_ref, lv_ref, sc_ref, mx_ref):
        i = pl.program_id(0)
        s2 = (r_ref[0] + r_ref[1]) * d_ref[...]
        mu = s2[:, :c_out]
        mu_ref[...] = mu
        lv_ref[...] = s2[:, c_out:]
        t = jnp.tanh(jnp.dot(mu, a1_ref[...], preferred_element_type=jnp.float32)
                     + ab1_ref[...])
        sc = jnp.dot(t, a2_ref[...], preferred_element_type=jnp.float32)
        sc_ref[...] = sc

        @pl.when(i == 0)
        def _():
            mx_ref[...] = jnp.full_like(mx_ref[...], -jnp.inf)
        mx_ref[...] = jnp.maximum(mx_ref[...], jnp.max(sc))

    return pl.pallas_call(
        body,
        grid=(n // block_rows,),
        in_specs=[
            pl.BlockSpec((2, block_rows, w), lambda i: (0, i, 0)),
            pl.BlockSpec((block_rows, 1), lambda i: (i, 0)),
            pl.BlockSpec((c_out, ah), lambda i: (0, 0)),
            pl.BlockSpec((1, ah), lambda i: (0, 0)),
            pl.BlockSpec((ah, 1), lambda i: (0, 0)),
        ],
        out_specs=[
            pl.BlockSpec((block_rows, c_out), lambda i: (i, 0)),
            pl.BlockSpec((block_rows, c_out), lambda i: (i, 0)),
            pl.BlockSpec((block_rows, 1), lambda i: (i, 0)),
            pl.BlockSpec((1, 1), lambda i: (0, 0)),
        ],
        out_shape=[
            jax.ShapeDtypeStruct((n, c_out), jnp.float32),
            jax.ShapeDtypeStruct((n, c_out), jnp.float32),
            jax.ShapeDtypeStruct((n, 1), jnp.float32),
            jax.ShapeDtypeStruct((1, 1), jnp.float32),
        ],
    )(rp, dv_is, a1, ab1.reshape(1, ah), a2)


def _shift_scores(scores, gmax):
    n = scores.shape[0]

    def body(s_ref, m_ref, o_ref):
        o_ref[...] = s_ref[...] - m_ref[0, 0]

    return pl.pallas_call(
        body,
        grid=(1,),
        in_specs=[
            pl.BlockSpec((n, 1), lambda i: (0, 0)),
            pl.BlockSpec((1, 1), lambda i: (0, 0)),
        ],
        out_specs=pl.BlockSpec((n, 1), lambda i: (0, 0)),
        out_shape=jax.ShapeDtypeStruct((n, 1), jnp.float32),
    )(scores, gmax)


def _rden_kernel(denp):
    def body(d_ref, o_ref):
        den = d_ref[0, :, 0:1] + d_ref[1, :, 0:1]
        o_ref[...] = 1.0 / jnp.maximum(den, 1e-12)

    return pl.pallas_call(
        body,
        grid=(1,),
        in_specs=[pl.BlockSpec((2, _E, _L), lambda i: (0, 0, 0))],
        out_specs=pl.BlockSpec((_E, 1), lambda i: (0, 0)),
        out_shape=jax.ShapeDtypeStruct((_E, 1), jnp.float32),
    )(denp)


def _decode(mu, zp, block_rows=1000):
    n, c = mu.shape

    def body(x_ref, z_ref, h_ref, z_out_ref):
        i = pl.program_id(0)
        z = z_ref[0] + z_ref[1]

        @pl.when(i == 0)
        def _():
            z_out_ref[...] = z
        acc = lax.dot_general(
            x_ref[...], z,
            dimension_numbers=(((1,), (1,)), ((), ())),
            preferred_element_type=jnp.float32)
        h_ref[...] = jax.nn.sigmoid(acc)

    return pl.pallas_call(
        body,
        grid=(n // block_rows,),
        in_specs=[
            pl.BlockSpec((block_rows, c), lambda i: (i, 0)),
            pl.BlockSpec((2, _E, c), lambda i: (0, 0, 0)),
        ],
        out_specs=[
            pl.BlockSpec((block_rows, _E), lambda i: (i, 0)),
            pl.BlockSpec((_E, c), lambda i: (0, 0)),
        ],
        out_shape=[
            jax.ShapeDtypeStruct((n, _E), jnp.float32),
            jax.ShapeDtypeStruct((_E, c), jnp.float32),
        ],
    )(mu, zp)


def kernel(X, vertex_idx, hyperedge_idx, W1, b1, W2, b2, W3, b3, A1, ab1, A2):
    c_out = W2.shape[1]
    pad = _NNZP - _NNZ

    vi = vertex_idx.astype(jnp.int32)
    ei = hyperedge_idx.astype(jnp.int32)
    v_g = jnp.concatenate([vi, jnp.zeros((pad,), jnp.int32)]).reshape(_ROWS, _CHA)
    v_s = jnp.concatenate([vi, jnp.full((pad,), _N, jnp.int32)]).reshape(_ROWS, _CHA)
    e_g = jnp.concatenate([ei, jnp.zeros((pad,), jnp.int32)]).reshape(_ROWS, _CHA)
    e_s = jnp.concatenate([ei, jnp.full((pad,), _E, jnp.int32)]).reshape(_ROWS, _CHA)

    ones_tab = jnp.ones((_CHA, _L), jnp.float32)
    z_n16 = jnp.zeros((_N + 1, _L), jnp.float32)
    z_e16 = jnp.zeros((_E + 1, _L), jnp.float32)
    z_e64 = jnp.zeros((_E + 1, 64), jnp.float32)
    z_n64 = jnp.zeros((_N + 1, 64), jnp.float32)
    z_row16 = jnp.zeros((_CHA, _L), jnp.float32)
    z_e32 = jnp.zeros((_E + 1, c_out), jnp.float32)

    dvp, dep = _sc_degrees(v_s, e_s, ones_tab, z_n16, z_e16)
    h0 = _mm_relu(X, W1, b1)

    wc = jnp.concatenate([W2, W3], axis=1)
    bc = jnp.concatenate([b2, b3], axis=0)
    x0, dv_is, de_inv = _deg_project(dvp, dep, h0, wc)

    yp = _sc_pair_pass(x0, v_g, e_s, _E, z_e64)
    ys = _comb_scale(yp, de_inv, _E)
    pp = _sc_pair_pass(ys, e_g, v_s, _N, z_n64)
    gs = _mid_project(pp, dv_is, bc)

    qp = _sc_pair_pass(gs, v_g, e_s, _E, z_e64)
    y2s = _comb_scale(qp, de_inv, _E)
    rp = _sc_pair_pass(y2s, e_g, v_s, _N, z_n64)

    mu, logvar, scores, gmax = _final_smooth_attention(
        rp, dv_is, A1, ab1, A2, c_out)

    ssh = _shift_scores(scores, gmax).reshape(_N)
    ex, denp = _sc_att_ex(ssh, v_g, e_s, z_row16, z_e16)
    rden = _rden_kernel(denp).reshape(_E)
    beta, zpart = _sc_att_z(ex, rden, mu, v_g, e_g, e_s, z_e32)
    h_out, z = _decode(mu, zpart)

    return (mu, z, h_out, mu, logvar, beta[:_NNZ])


# speedup vs baseline: 20.4728x; 1.3980x over previous
import functools

import jax
import jax.numpy as jnp
from jax import lax
from jax.experimental import pallas as pl
from jax.experimental.pallas import tpu as pltpu
from jax.experimental.pallas import tpu_sc as plsc

_NC = 2
_NS = 16
_L = 16
_NW = _NC * _NS

_N = 10000
_E = 2000
_NNZ = 320000

_CHA = 64
_CHT = 160
_PT = _CHA * _CHT
_NNZP = _PT * _NW
_ROWS = _NNZP // _CHA
_R = 4


def _sc_mesh():
    return plsc.VectorSubcoreMesh(
        core_axis_name="c", subcore_axis_name="s",
        num_cores=_NC, num_subcores=_NS)


_SC_PARAMS = pltpu.CompilerParams(
    use_tc_tiling_on_sc=False, needs_layout_passes=False)


def _sc_degrees(v_s, e_s, ones_tab, zv, ze):
    @functools.partial(
        pl.kernel,
        out_type=(jax.ShapeDtypeStruct((_NC, _N + 1, _L), jnp.float32),
                  jax.ShapeDtypeStruct((_NC, _E + 1, _L), jnp.float32)),
        mesh=_sc_mesh(),
        compiler_params=_SC_PARAMS,
        scratch_types=[
            pltpu.VMEM((_CHT, _CHA), jnp.int32),
            pltpu.VMEM((_CHT, _CHA), jnp.int32),
            pltpu.VMEM((_CHA, _L), jnp.float32),
            pltpu.VMEM_SHARED((_N + 1, _L), jnp.float32),
            pltpu.VMEM_SHARED((_E + 1, _L), jnp.float32),
            pltpu.SemaphoreType.DMA,
            pltpu.SemaphoreType.DMA,
        ],
    )
    def kfn(v2_h, e2_h, ones_h, zv_h, ze_h, dvp_h, dep_h,
            vbuf, ebuf, ones_v, accv, acce, semv, seme):
        c = lax.axis_index("c")
        s = lax.axis_index("s")
        wid = c * _NS + s
        pltpu.sync_copy(v2_h.at[pl.ds(wid * _CHT, _CHT)], vbuf)
        pltpu.sync_copy(e2_h.at[pl.ds(wid * _CHT, _CHT)], ebuf)
        pltpu.sync_copy(ones_h, ones_v)

        @pl.when(s == 0)
        def _():
            pltpu.sync_copy(zv_h, accv)
            pltpu.sync_copy(ze_h, acce)
        plsc.subcore_barrier()

        @pl.loop(0, _CHT)
        def _(j):
            @pl.when(j >= _R)
            def _():
                pltpu.make_async_copy(
                    ones_v, accv.at[vbuf.at[j - _R]], semv).wait()
                pltpu.make_async_copy(
                    ones_v, acce.at[ebuf.at[j - _R]], seme).wait()
            pltpu.async_copy(ones_v, accv.at[vbuf.at[j]], semv, add=True)
            pltpu.async_copy(ones_v, acce.at[ebuf.at[j]], seme, add=True)

        for t in range(_R):
            pltpu.make_async_copy(
                ones_v, accv.at[vbuf.at[_CHT - _R + t]], semv).wait()
            pltpu.make_async_copy(
                ones_v, acce.at[ebuf.at[_CHT - _R + t]], seme).wait()

        plsc.subcore_barrier()

        @pl.when(s == 0)
        def _():
            pltpu.sync_copy(accv, dvp_h.at[c])
            pltpu.sync_copy(acce, dep_h.at[c])

    return kfn(v_s, e_s, ones_tab, zv, ze)


def _sc_pair_pass(table, g2, s2, out_rows, zeros_acc):
    rt, w = table.shape

    @functools.partial(
        pl.kernel,
        out_type=jax.ShapeDtypeStruct((_NC, out_rows + 1, w), jnp.float32),
        mesh=_sc_mesh(),
        compiler_params=_SC_PARAMS,
        scratch_types=[
            pltpu.VMEM((_CHT, _CHA), jnp.int32),
            pltpu.VMEM((_CHT, _CHA), jnp.int32),
            [pltpu.VMEM((_CHA, w), jnp.float32) for _ in range(_R)],
            pltpu.VMEM_SHARED((rt, w), jnp.float32),
            pltpu.VMEM_SHARED((out_rows + 1, w), jnp.float32),
            [pltpu.SemaphoreType.DMA for _ in range(_R)],
            [pltpu.SemaphoreType.DMA for _ in range(_R)],
        ],
    )
    def kfn(tab_h, g2_h, s2_h, zz_h, out_h,
            gbuf, sbuf, rows, tab_sh, acc, gsems, ssems):
        c = lax.axis_index("c")
        s = lax.axis_index("s")
        wid = c * _NS + s
        pltpu.sync_copy(g2_h.at[pl.ds(wid * _CHT, _CHT)], gbuf)
        pltpu.sync_copy(s2_h.at[pl.ds(wid * _CHT, _CHT)], sbuf)

        @pl.when(s == 0)
        def _():
            pltpu.sync_copy(zz_h, acc)

        @pl.when(s == 1)
        def _():
            pltpu.sync_copy(tab_h, tab_sh)
        plsc.subcore_barrier()

        pltpu.async_copy(tab_sh.at[gbuf.at[0]], rows[0], gsems[0])
        pltpu.async_copy(tab_sh.at[gbuf.at[1]], rows[1], gsems[1])

        @pl.loop(0, _CHT, step=_R)
        def _(j):
            for b in range(_R):
                k = j + b
                bp = (b + 2) % _R
                pltpu.make_async_copy(
                    tab_sh.at[gbuf.at[k]], rows[b], gsems[b]).wait()
                pltpu.async_copy(
                    rows[b], acc.at[sbuf.at[k]], ssems[b], add=True)

                @pl.when(k >= 2)
                def _():
                    pltpu.make_async_copy(
                        rows[bp], acc.at[sbuf.at[k - 2]], ssems[bp]).wait()

                @pl.when(k + 2 < _CHT)
                def _():
                    pltpu.async_copy(
                        tab_sh.at[gbuf.at[k + 2]], rows[bp], gsems[bp])

        for k in (_CHT - 2, _CHT - 1):
            pltpu.make_async_copy(
                rows[k % _R], acc.at[sbuf.at[k]], ssems[k % _R]).wait()

        plsc.subcore_barrier()

        @pl.when(s == 0)
        def _():
            pltpu.sync_copy(acc, out_h.at[c])

    return kfn(table, g2, s2, zeros_acc)


def _sc_att_ex(ssh, v_g, e_s, zrow, zden):
    @functools.partial(
        pl.kernel,
        out_type=(jax.ShapeDtypeStruct((_NNZP,), jnp.float32),
                  jax.ShapeDtypeStruct((_NC, _E + 1, _L), jnp.float32)),
        mesh=_sc_mesh(),
        compiler_params=_SC_PARAMS,
        scratch_types=[
            pltpu.VMEM((_CHT, _CHA), jnp.int32),
            pltpu.VMEM((_CHT, _CHA), jnp.int32),
            pltpu.VMEM((_N,), jnp.float32),
            pltpu.VMEM((_PT,), jnp.float32),
            [pltpu.VMEM((_CHA, _L), jnp.float32) for _ in range(_R)],
            pltpu.VMEM_SHARED((_E + 1, _L), jnp.float32),
            [pltpu.SemaphoreType.DMA for _ in range(_R)],
        ],
    )
    def kfn(ssh_h, v2_h, e2_h, zrow_h, zden_h, ex_h, denp_h,
            vbuf, ebuf, stab, extile, exbufs, accd, ssems):
        c = lax.axis_index("c")
        s = lax.axis_index("s")
        wid = c * _NS + s
        pltpu.sync_copy(v2_h.at[pl.ds(wid * _CHT, _CHT)], vbuf)
        pltpu.sync_copy(e2_h.at[pl.ds(wid * _CHT, _CHT)], ebuf)
        pltpu.sync_copy(ssh_h, stab)
        for b in range(_R):
            pltpu.sync_copy(zrow_h, exbufs[b])

        @pl.when(s == 0)
        def _():
            pltpu.sync_copy(zden_h, accd)
        plsc.subcore_barrier()

        @pl.loop(0, _CHT, step=_R)
        def _(j):
            for b in range(_R):
                k = j + b

                @pl.when(k >= _R)
                def _():
                    pltpu.make_async_copy(
                        exbufs[b], accd.at[ebuf.at[k - _R]], ssems[b]).wait()
                for u in range(_CHA // _L):
                    idx = vbuf[k, pl.ds(u * _L, _L)]
                    ex = jnp.exp(plsc.load_gather(stab, [idx]))
                    extile[pl.ds(k * _CHA + u * _L, _L)] = ex
                    lane = lax.iota(jnp.int32, _L)
                    plsc.store_scatter(exbufs[b], [lane + (u * _L), lane], ex)
                pltpu.async_copy(
                    exbufs[b], accd.at[ebuf.at[k]], ssems[b], add=True)

        for b in range(_R):
            pltpu.make_async_copy(
                exbufs[b], accd.at[ebuf.at[_CHT - _R + b]], ssems[b]).wait()

        plsc.subcore_barrier()
        pltpu.sync_copy(extile, ex_h.at[pl.ds(wid * _PT, _PT)])

        @pl.when(s == 0)
        def _():
            pltpu.sync_copy(accd, denp_h.at[c])

    return kfn(ssh, v_g, e_s, zrow, zden)


def _sc_att_z(ex, rden, mu, v_g, e_g, e_s, zz):
    c_out = mu.shape[1]

    @functools.partial(
        pl.kernel,
        out_type=(jax.ShapeDtypeStruct((_NNZP,), jnp.float32),
                  jax.ShapeDtypeStruct((_NC, _E + 1, c_out), jnp.float32)),
        mesh=_sc_mesh(),
        compiler_params=_SC_PARAMS,
        scratch_types=[
            pltpu.VMEM((_CHT, _CHA), jnp.int32),
            pltpu.VMEM((_CHT, _CHA), jnp.int32),
            pltpu.VMEM((_CHT, _CHA), jnp.int32),
            pltpu.VMEM((_E,), jnp.float32),
            pltpu.VMEM((_PT,), jnp.float32),
            pltpu.VMEM((_PT,), jnp.float32),
            [pltpu.VMEM((_CHA, c_out), jnp.float32) for _ in range(_R)],
            pltpu.VMEM_SHARED((_N, c_out), jnp.float32),
            pltpu.VMEM_SHARED((_E + 1, c_out), jnp.float32),
            [pltpu.SemaphoreType.DMA for _ in range(_R)],
            [pltpu.SemaphoreType.DMA for _ in range(_R)],
        ],
    )
    def kfn(ex_h, rden_h, mu_h, vg_h, eg_h, es_h, zz_h, beta_h, zp_h,
            vbuf, egbuf, esbuf, rdtab, extile, betatile, rows,
            mu_sh, accz, gsems, ssems):
        c = lax.axis_index("c")
        s = lax.axis_index("s")
        wid = c * _NS + s
        pltpu.sync_copy(vg_h.at[pl.ds(wid * _CHT, _CHT)], vbuf)
        pltpu.sync_copy(eg_h.at[pl.ds(wid * _CHT, _CHT)], egbuf)
        pltpu.sync_copy(es_h.at[pl.ds(wid * _CHT, _CHT)], esbuf)
        pltpu.sync_copy(rden_h, rdtab)
        pltpu.sync_copy(ex_h.at[pl.ds(wid * _PT, _PT)], extile)

        @pl.when(s == 0)
        def _():
            pltpu.sync_copy(zz_h, accz)

        @pl.when(s == 1)
        def _():
            pltpu.sync_copy(mu_h, mu_sh)
        plsc.subcore_barrier()

        pltpu.async_copy(mu_sh.at[vbuf.at[0]], rows[0], gsems[0])
        pltpu.async_copy(mu_sh.at[vbuf.at[1]], rows[1], gsems[1])

        @pl.loop(0, _CHT, step=_R)
        def _(j):
            for b in range(_R):
                k = j + b
                bp = (b + 2) % _R
                pltpu.make_async_copy(
                    mu_sh.at[vbuf.at[k]], rows[b], gsems[b]).wait()
                for u in range(_CHA // _L):
                    eidx = egbuf[k, pl.ds(u * _L, _L)]
                    rd = plsc.load_gather(rdtab, [eidx])
                    exv = extile[pl.ds(k * _CHA + u * _L, _L)]
                    bet = exv * rd
                    betatile[pl.ds(k * _CHA + u * _L, _L)] = bet
                    lane = lax.iota(jnp.int32, _L)
                    rowi = lane + (u * _L)
                    for col in range(c_out):
                        ci = lax.rem(lane + col, c_out)
                        vals = plsc.load_gather(rows[b], [rowi, ci])
                        plsc.store_scatter(rows[b], [rowi, ci], vals * bet)
                pltpu.async_copy(
                    rows[b], accz.at[esbuf.at[k]], ssems[b], add=True)

                @pl.when(k >= 2)
                def _():
                    pltpu.make_async_copy(
                        rows[bp], accz.at[esbuf.at[k - 2]], ssems[bp]).wait()

                @pl.when(k + 2 < _CHT)
                def _():
                    pltpu.async_copy(
                        mu_sh.at[vbuf.at[k + 2]], rows[bp], gsems[bp])

        for k in (_CHT - 2, _CHT - 1):
            pltpu.make_async_copy(
                rows[k % _R], accz.at[esbuf.at[k]], ssems[k % _R]).wait()

        plsc.subcore_barrier()
        pltpu.sync_copy(betatile, beta_h.at[pl.ds(wid * _PT, _PT)])

        @pl.when(s == 0)
        def _():
            pltpu.sync_copy(accz, zp_h.at[c])

    return kfn(ex, rden, mu, v_g, e_g, e_s, zz)


def _mm_relu(x, w, b, block_rows=1000):
    n, k = x.shape
    _, m = w.shape

    def body(x_ref, w_ref, b_ref, o_ref):
        acc = jnp.dot(x_ref[...], w_ref[...], preferred_element_type=jnp.float32)
        o_ref[...] = jax.nn.relu(acc + b_ref[...])

    return pl.pallas_call(
        body,
        grid=(n // block_rows,),
        in_specs=[
            pl.BlockSpec((block_rows, k), lambda i: (i, 0)),
            pl.BlockSpec((k, m), lambda i: (0, 0)),
            pl.BlockSpec((1, m), lambda i: (0, 0)),
        ],
        out_specs=pl.BlockSpec((block_rows, m), lambda i: (i, 0)),
        out_shape=jax.ShapeDtypeStruct((n, m), jnp.float32),
    )(x, w, b.reshape(1, m))


def _deg_project(dvp, dep, h0, wc, block_rows=1000):
    n = h0.shape[0]
    cin = h0.shape[1]
    m = wc.shape[1]

    def body(dv_ref, de_ref, h_ref, w_ref, x0_ref, dvis_ref, dei_ref):
        i = pl.program_id(0)
        dv = dv_ref[0, :, 0:1] + dv_ref[1, :, 0:1]
        dvis = jnp.where(dv > 0, lax.rsqrt(jnp.maximum(dv, 1e-12)), 0.0)
        dvis_ref[...] = dvis
        x0_ref[...] = jnp.dot(h_ref[...] * dvis, w_ref[...],
                              preferred_element_type=jnp.float32)

        @pl.when(i == 0)
        def _():
            de = de_ref[0, :, 0:1] + de_ref[1, :, 0:1]
            dei_ref[...] = jnp.where(de > 0, 1.0 / jnp.maximum(de, 1e-12), 0.0)

    return pl.pallas_call(
        body,
        grid=(n // block_rows,),
        in_specs=[
            pl.BlockSpec((2, block_rows, _L), lambda i: (0, i, 0)),
            pl.BlockSpec((2, _E, _L), lambda i: (0, 0, 0)),
            pl.BlockSpec((block_rows, cin), lambda i: (i, 0)),
            pl.BlockSpec((cin, m), lambda i: (0, 0)),
        ],
        out_specs=[
            pl.BlockSpec((block_rows, m), lambda i: (i, 0)),
            pl.BlockSpec((block_rows, 1), lambda i: (i, 0)),
            pl.BlockSpec((_E, 1), lambda i: (0, 0)),
        ],
        out_shape=[
            jax.ShapeDtypeStruct((n, m), jnp.float32),
            jax.ShapeDtypeStruct((n, 1), jnp.float32),
            jax.ShapeDtypeStruct((_E, 1), jnp.float32),
        ],
    )(dvp, dep, h0, wc)


def _comb_scale(yp, scale, rows):
    _, _, w = yp.shape

    def body(y_ref, s_ref, o_ref):
        o_ref[...] = (y_ref[0] + y_ref[1]) * s_ref[...]

    return pl.pallas_call(
        body,
        grid=(1,),
        in_specs=[
            pl.BlockSpec((2, rows, w), lambda i: (0, 0, 0)),
            pl.BlockSpec((rows, 1), lambda i: (0, 0)),
        ],
        out_specs=pl.BlockSpec((rows, w), lambda i: (0, 0)),
        out_shape=jax.ShapeDtypeStruct((rows, w), jnp.float32),
    )(yp, scale)


def _mid_project(pp, dv_is, bc, block_rows=1000):
    _, _, w = pp.shape
    n = dv_is.shape[0]

    def body(p_ref, d_ref, b_ref, o_ref):
        d = d_ref[...]
        o_ref[...] = d * d * (p_ref[0] + p_ref[1]) + d * b_ref[...]

    return pl.pallas_call(
        body,
        grid=(n // block_rows,),
        in_specs=[
            pl.BlockSpec((2, block_rows, w), lambda i: (0, i, 0)),
            pl.BlockSpec((block_rows, 1), lambda i: (i, 0)),
            pl.BlockSpec((1, w), lambda i: (0, 0)),
        ],
        out_specs=pl.BlockSpec((block_rows, w), lambda i: (i, 0)),
        out_shape=jax.ShapeDtypeStruct((n, w), jnp.float32),
    )(pp, dv_is, bc.reshape(1, w))


def _final_smooth_attention(rp, dv_is, a1, ab1, a2, c_out, block_rows=1000):
    _, _, w = rp.shape
    n = dv_is.shape[0]
    ah = a1.shape[1]

    def body(r_ref, d_ref, a1_ref, ab1_ref, a2_ref,
             mu_ref, lv_ref, sc_ref, mx_ref):
        i = pl.program_id(0)
        s2 = (r_ref[0] + r_ref[1]) * d_ref[...]
        mu = s2[:, :c_out]
        mu_ref[...] = mu
        lv_ref[...] = s2[:, c_out:]
        t = jnp.tanh(jnp.dot(mu, a1_ref[...], preferred_element_type=jnp.float32)
                     + ab1_ref[...])
        sc = jnp.dot(t, a2_ref[...], preferred_element_type=jnp.float32)
        sc_ref[...] = sc

        @pl.when(i == 0)
        def _():
            mx_ref[...] = jnp.full_like(mx_ref[...], -jnp.inf)
        mx_ref[...] = jnp.maximum(mx_ref[...], jnp.max(sc))

    return pl.pallas_call(
        body,
        grid=(n // block_rows,),
        in_specs=[
            pl.BlockSpec((2, block_rows, w), lambda i: (0, i, 0)),
            pl.BlockSpec((block_rows, 1), lambda i: (i, 0)),
            pl.BlockSpec((c_out, ah), lambda i: (0, 0)),
            pl.BlockSpec((1, ah), lambda i: (0, 0)),
            pl.BlockSpec((ah, 1), lambda i: (0, 0)),
        ],
        out_specs=[
            pl.BlockSpec((block_rows, c_out), lambda i: (i, 0)),
            pl.BlockSpec((block_rows, c_out), lambda i: (i, 0)),
            pl.BlockSpec((block_rows, 1), lambda i: (i, 0)),
            pl.BlockSpec((1, 1), lambda i: (0, 0)),
        ],
        out_shape=[
            jax.ShapeDtypeStruct((n, c_out), jnp.float32),
            jax.ShapeDtypeStruct((n, c_out), jnp.float32),
            jax.ShapeDtypeStruct((n, 1), jnp.float32),
            jax.ShapeDtypeStruct((1, 1), jnp.float32),
        ],
    )(rp, dv_is, a1, ab1.reshape(1, ah), a2)


def _shift_scores(scores, gmax):
    n = scores.shape[0]

    def body(s_ref, m_ref, o_ref):
        o_ref[...] = s_ref[...] - m_ref[0, 0]

    return pl.pallas_call(
        body,
        grid=(1,),
        in_specs=[
            pl.BlockSpec((n, 1), lambda i: (0, 0)),
            pl.BlockSpec((1, 1), lambda i: (0, 0)),
        ],
        out_specs=pl.BlockSpec((n, 1), lambda i: (0, 0)),
        out_shape=jax.ShapeDtypeStruct((n, 1), jnp.float32),
    )(scores, gmax)


def _rden_kernel(denp):
    def body(d_ref, o_ref):
        den = jnp.sum(d_ref[0] + d_ref[1], axis=-1, keepdims=True)
        o_ref[...] = 1.0 / jnp.maximum(den, 1e-12)

    return pl.pallas_call(
        body,
        grid=(1,),
        in_specs=[pl.BlockSpec((2, _E, _L), lambda i: (0, 0, 0))],
        out_specs=pl.BlockSpec((_E, 1), lambda i: (0, 0)),
        out_shape=jax.ShapeDtypeStruct((_E, 1), jnp.float32),
    )(denp)


def _decode(mu, zp, block_rows=1000):
    n, c = mu.shape

    def body(x_ref, z_ref, h_ref, z_out_ref):
        i = pl.program_id(0)
        z = z_ref[0] + z_ref[1]

        @pl.when(i == 0)
        def _():
            z_out_ref[...] = z
        acc = lax.dot_general(
            x_ref[...], z,
            dimension_numbers=(((1,), (1,)), ((), ())),
            preferred_element_type=jnp.float32)
        h_ref[...] = jax.nn.sigmoid(acc)

    return pl.pallas_call(
        body,
        grid=(n // block_rows,),
        in_specs=[
            pl.BlockSpec((block_rows, c), lambda i: (i, 0)),
            pl.BlockSpec((2, _E, c), lambda i: (0, 0, 0)),
        ],
        out_specs=[
            pl.BlockSpec((block_rows, _E), lambda i: (i, 0)),
            pl.BlockSpec((_E, c), lambda i: (0, 0)),
        ],
        out_shape=[
            jax.ShapeDtypeStruct((n, _E), jnp.float32),
            jax.ShapeDtypeStruct((_E, c), jnp.float32),
        ],
    )(mu, zp)


def kernel(X, vertex_idx, hyperedge_idx, W1, b1, W2, b2, W3, b3, A1, ab1, A2):
    c_out = W2.shape[1]
    pad = _NNZP - _NNZ

    vi = vertex_idx.astype(jnp.int32)
    ei = hyperedge_idx.astype(jnp.int32)
    v_g = jnp.concatenate([vi, jnp.zeros((pad,), jnp.int32)]).reshape(_ROWS, _CHA)
    v_s = jnp.concatenate([vi, jnp.full((pad,), _N, jnp.int32)]).reshape(_ROWS, _CHA)
    e_g = jnp.concatenate([ei, jnp.zeros((pad,), jnp.int32)]).reshape(_ROWS, _CHA)
    e_s = jnp.concatenate([ei, jnp.full((pad,), _E, jnp.int32)]).reshape(_ROWS, _CHA)

    ones_tab = jnp.ones((_CHA, _L), jnp.float32)
    z_n16 = jnp.zeros((_N + 1, _L), jnp.float32)
    z_e16 = jnp.zeros((_E + 1, _L), jnp.float32)
    z_e64 = jnp.zeros((_E + 1, 64), jnp.float32)
    z_n64 = jnp.zeros((_N + 1, 64), jnp.float32)
    z_row16 = jnp.zeros((_CHA, _L), jnp.float32)
    z_e32 = jnp.zeros((_E + 1, c_out), jnp.float32)

    dvp, dep = _sc_degrees(v_s, e_s, ones_tab, z_n16, z_e16)
    h0 = _mm_relu(X, W1, b1)

    wc = jnp.concatenate([W2, W3], axis=1)
    bc = jnp.concatenate([b2, b3], axis=0)
    x0, dv_is, de_inv = _deg_project(dvp, dep, h0, wc)

    yp = _sc_pair_pass(x0, v_g, e_s, _E, z_e64)
    ys = _comb_scale(yp, de_inv, _E)
    pp = _sc_pair_pass(ys, e_g, v_s, _N, z_n64)
    gs = _mid_project(pp, dv_is, bc)

    qp = _sc_pair_pass(gs, v_g, e_s, _E, z_e64)
    y2s = _comb_scale(qp, de_inv, _E)
    rp = _sc_pair_pass(y2s, e_g, v_s, _N, z_n64)

    mu, logvar, scores, gmax = _final_smooth_attention(
        rp, dv_is, A1, ab1, A2, c_out)

    ssh = _shift_scores(scores, gmax).reshape(_N)
    ex, denp = _sc_att_ex(ssh, v_g, e_s, z_row16, z_e16)
    rden = _rden_kernel(denp).reshape(_E)
    beta, zpart = _sc_att_z(ex, rden, mu, v_g, e_g, e_s, z_e32)
    h_out, z = _decode(mu, zpart)

    return (mu, z, h_out, mu, logvar, beta[:_NNZ])


# speedup vs baseline: 20.6856x; 1.0104x over previous
import functools

import jax
import jax.numpy as jnp
from jax import lax
from jax.experimental import pallas as pl
from jax.experimental.pallas import tpu as pltpu
from jax.experimental.pallas import tpu_sc as plsc

_NC = 2
_NS = 16
_L = 16
_NW = _NC * _NS

_N = 10000
_E = 2000
_NNZ = 320000

_CHA = 64
_CHT = 160
_PT = _CHA * _CHT
_NNZP = _PT * _NW
_ROWS = _NNZP // _CHA
_R = 4


def _sc_mesh():
    return plsc.VectorSubcoreMesh(
        core_axis_name="c", subcore_axis_name="s",
        num_cores=_NC, num_subcores=_NS)


_SC_PARAMS = pltpu.CompilerParams(
    use_tc_tiling_on_sc=False, needs_layout_passes=False)


def _sc_degrees(v_s, e_s, ones_tab, zv, ze):
    @functools.partial(
        pl.kernel,
        out_type=(jax.ShapeDtypeStruct((_NC, _N + 1, _L), jnp.float32),
                  jax.ShapeDtypeStruct((_NC, _E + 1, _L), jnp.float32)),
        mesh=_sc_mesh(),
        compiler_params=_SC_PARAMS,
        scratch_types=[
            pltpu.VMEM((_CHT, _CHA), jnp.int32),
            pltpu.VMEM((_CHT, _CHA), jnp.int32),
            pltpu.VMEM((_CHA, _L), jnp.float32),
            pltpu.VMEM_SHARED((_N + 1, _L), jnp.float32),
            pltpu.VMEM_SHARED((_E + 1, _L), jnp.float32),
            pltpu.SemaphoreType.DMA,
            pltpu.SemaphoreType.DMA,
        ],
    )
    def kfn(v2_h, e2_h, ones_h, zv_h, ze_h, dvp_h, dep_h,
            vbuf, ebuf, ones_v, accv, acce, semv, seme):
        c = lax.axis_index("c")
        s = lax.axis_index("s")
        wid = c * _NS + s
        pltpu.sync_copy(v2_h.at[pl.ds(wid * _CHT, _CHT)], vbuf)
        pltpu.sync_copy(e2_h.at[pl.ds(wid * _CHT, _CHT)], ebuf)
        pltpu.sync_copy(ones_h, ones_v)

        @pl.when(s == 0)
        def _():
            pltpu.sync_copy(zv_h, accv)
            pltpu.sync_copy(ze_h, acce)
        plsc.subcore_barrier()

        @pl.loop(0, _CHT)
        def _(j):
            @pl.when(j >= _R)
            def _():
                pltpu.make_async_copy(
                    ones_v, accv.at[vbuf.at[j - _R]], semv).wait()
                pltpu.make_async_copy(
                    ones_v, acce.at[ebuf.at[j - _R]], seme).wait()
            pltpu.async_copy(ones_v, accv.at[vbuf.at[j]], semv, add=True)
            pltpu.async_copy(ones_v, acce.at[ebuf.at[j]], seme, add=True)

        for t in range(_R):
            pltpu.make_async_copy(
                ones_v, accv.at[vbuf.at[_CHT - _R + t]], semv).wait()
            pltpu.make_async_copy(
                ones_v, acce.at[ebuf.at[_CHT - _R + t]], seme).wait()

        plsc.subcore_barrier()

        @pl.when(s == 0)
        def _():
            pltpu.sync_copy(accv, dvp_h.at[c])
            pltpu.sync_copy(acce, dep_h.at[c])

    return kfn(v_s, e_s, ones_tab, zv, ze)


def _sc_pair_pass(table, g2, s2, out_rows, zeros_acc):
    rt, w = table.shape

    @functools.partial(
        pl.kernel,
        out_type=jax.ShapeDtypeStruct((_NC, out_rows + 1, w), jnp.float32),
        mesh=_sc_mesh(),
        compiler_params=_SC_PARAMS,
        scratch_types=[
            pltpu.VMEM((_CHT, _CHA), jnp.int32),
            pltpu.VMEM((_CHT, _CHA), jnp.int32),
            [pltpu.VMEM((_CHA, w), jnp.float32) for _ in range(_R)],
            pltpu.VMEM_SHARED((rt, w), jnp.float32),
            pltpu.VMEM_SHARED((out_rows + 1, w), jnp.float32),
            [pltpu.SemaphoreType.DMA for _ in range(_R)],
            [pltpu.SemaphoreType.DMA for _ in range(_R)],
        ],
    )
    def kfn(tab_h, g2_h, s2_h, zz_h, out_h,
            gbuf, sbuf, rows, tab_sh, acc, gsems, ssems):
        c = lax.axis_index("c")
        s = lax.axis_index("s")
        wid = c * _NS + s
        pltpu.sync_copy(g2_h.at[pl.ds(wid * _CHT, _CHT)], gbuf)
        pltpu.sync_copy(s2_h.at[pl.ds(wid * _CHT, _CHT)], sbuf)

        @pl.when(s == 0)
        def _():
            pltpu.sync_copy(zz_h, acc)

        @pl.when(s == 1)
        def _():
            pltpu.sync_copy(tab_h, tab_sh)
        plsc.subcore_barrier()

        pltpu.async_copy(tab_sh.at[gbuf.at[0]], rows[0], gsems[0])
        pltpu.async_copy(tab_sh.at[gbuf.at[1]], rows[1], gsems[1])

        @pl.loop(0, _CHT, step=_R)
        def _(j):
            for b in range(_R):
                k = j + b
                bp = (b + 2) % _R
                pltpu.make_async_copy(
                    tab_sh.at[gbuf.at[k]], rows[b], gsems[b]).wait()
                pltpu.async_copy(
                    rows[b], acc.at[sbuf.at[k]], ssems[b], add=True)

                @pl.when(k >= 2)
                def _():
                    pltpu.make_async_copy(
                        rows[bp], acc.at[sbuf.at[k - 2]], ssems[bp]).wait()

                @pl.when(k + 2 < _CHT)
                def _():
                    pltpu.async_copy(
                        tab_sh.at[gbuf.at[k + 2]], rows[bp], gsems[bp])

        for k in (_CHT - 2, _CHT - 1):
            pltpu.make_async_copy(
                rows[k % _R], acc.at[sbuf.at[k]], ssems[k % _R]).wait()

        plsc.subcore_barrier()

        @pl.when(s == 0)
        def _():
            pltpu.sync_copy(acc, out_h.at[c])

    return kfn(table, g2, s2, zeros_acc)


def _sc_att_ex(ssh, v_g, e_s, zrow, zden):
    @functools.partial(
        pl.kernel,
        out_type=(jax.ShapeDtypeStruct((_NNZP,), jnp.float32),
                  jax.ShapeDtypeStruct((_NC, _E + 1, _L), jnp.float32)),
        mesh=_sc_mesh(),
        compiler_params=_SC_PARAMS,
        scratch_types=[
            pltpu.VMEM((_CHT, _CHA), jnp.int32),
            pltpu.VMEM((_CHT, _CHA), jnp.int32),
            pltpu.VMEM((_N,), jnp.float32),
            pltpu.VMEM((_PT,), jnp.float32),
            [pltpu.VMEM((_CHA, _L), jnp.float32) for _ in range(_R)],
            pltpu.VMEM_SHARED((_E + 1, _L), jnp.float32),
            [pltpu.SemaphoreType.DMA for _ in range(_R)],
        ],
    )
    def kfn(ssh_h, v2_h, e2_h, zrow_h, zden_h, ex_h, denp_h,
            vbuf, ebuf, stab, extile, exbufs, accd, ssems):
        c = lax.axis_index("c")
        s = lax.axis_index("s")
        wid = c * _NS + s
        pltpu.sync_copy(v2_h.at[pl.ds(wid * _CHT, _CHT)], vbuf)
        pltpu.sync_copy(e2_h.at[pl.ds(wid * _CHT, _CHT)], ebuf)
        pltpu.sync_copy(ssh_h, stab)
        for b in range(_R):
            pltpu.sync_copy(zrow_h, exbufs[b])

        @pl.when(s == 0)
        def _():
            pltpu.sync_copy(zden_h, accd)
        plsc.subcore_barrier()

        @pl.loop(0, _CHT, step=_R)
        def _(j):
            for b in range(_R):
                k = j + b

                @pl.when(k >= _R)
                def _():
                    pltpu.make_async_copy(
                        exbufs[b], accd.at[ebuf.at[k - _R]], ssems[b]).wait()
                for u in range(_CHA // _L):
                    idx = vbuf[k, pl.ds(u * _L, _L)]
                    ex = jnp.exp(plsc.load_gather(stab, [idx]))
                    extile[pl.ds(k * _CHA + u * _L, _L)] = ex
                    lane = lax.iota(jnp.int32, _L)
                    plsc.store_scatter(exbufs[b], [lane + (u * _L), lane], ex)
                pltpu.async_copy(
                    exbufs[b], accd.at[ebuf.at[k]], ssems[b], add=True)

        for b in range(_R):
            pltpu.make_async_copy(
                exbufs[b], accd.at[ebuf.at[_CHT - _R + b]], ssems[b]).wait()

        plsc.subcore_barrier()
        pltpu.sync_copy(extile, ex_h.at[pl.ds(wid * _PT, _PT)])

        @pl.when(s == 0)
        def _():
            pltpu.sync_copy(accd, denp_h.at[c])

    return kfn(ssh, v_g, e_s, zrow, zden)


def _sc_att_z(ex, rden, mu, v_g, e_g, e_s, zz):
    c_out = mu.shape[1]
    rz = 8
    la = 4

    @functools.partial(
        pl.kernel,
        out_type=(jax.ShapeDtypeStruct((_NNZP,), jnp.float32),
                  jax.ShapeDtypeStruct((_NC, _E + 1, c_out), jnp.float32)),
        mesh=_sc_mesh(),
        compiler_params=_SC_PARAMS,
        scratch_types=[
            pltpu.VMEM((_CHT, _CHA), jnp.int32),
            pltpu.VMEM((_CHT, _CHA), jnp.int32),
            pltpu.VMEM((_CHT, _CHA), jnp.int32),
            pltpu.VMEM((_E,), jnp.float32),
            pltpu.VMEM((_PT,), jnp.float32),
            pltpu.VMEM((_PT,), jnp.float32),
            [pltpu.VMEM((_CHA, c_out), jnp.float32) for _ in range(rz)],
            pltpu.VMEM_SHARED((_N, c_out), jnp.float32),
            pltpu.VMEM_SHARED((_E + 1, c_out), jnp.float32),
            [pltpu.SemaphoreType.DMA for _ in range(rz)],
            [pltpu.SemaphoreType.DMA for _ in range(rz)],
        ],
    )
    def kfn(ex_h, rden_h, mu_h, vg_h, eg_h, es_h, zz_h, beta_h, zp_h,
            vbuf, egbuf, esbuf, rdtab, extile, betatile, rows,
            mu_sh, accz, gsems, ssems):
        c = lax.axis_index("c")
        s = lax.axis_index("s")
        wid = c * _NS + s
        pltpu.sync_copy(vg_h.at[pl.ds(wid * _CHT, _CHT)], vbuf)
        pltpu.sync_copy(eg_h.at[pl.ds(wid * _CHT, _CHT)], egbuf)
        pltpu.sync_copy(es_h.at[pl.ds(wid * _CHT, _CHT)], esbuf)
        pltpu.sync_copy(rden_h, rdtab)
        pltpu.sync_copy(ex_h.at[pl.ds(wid * _PT, _PT)], extile)

        @pl.when(s == 0)
        def _():
            pltpu.sync_copy(zz_h, accz)

        @pl.when(s == 1)
        def _():
            pltpu.sync_copy(mu_h, mu_sh)
        plsc.subcore_barrier()

        for t in range(la):
            pltpu.async_copy(mu_sh.at[vbuf.at[t]], rows[t], gsems[t])

        @pl.loop(0, _CHT, step=rz)
        def _(j):
            for b in range(rz):
                k = j + b
                bp = (b + la) % rz
                pltpu.make_async_copy(
                    mu_sh.at[vbuf.at[k]], rows[b], gsems[b]).wait()
                for u in range(_CHA // _L):
                    eidx = egbuf[k, pl.ds(u * _L, _L)]
                    rd = plsc.load_gather(rdtab, [eidx])
                    exv = extile[pl.ds(k * _CHA + u * _L, _L)]
                    bet = exv * rd
                    betatile[pl.ds(k * _CHA + u * _L, _L)] = bet
                    lane = lax.iota(jnp.int32, _L)
                    rowi = lane + (u * _L)
                    for col in range(c_out):
                        ci = lax.rem(lane + col, c_out)
                        vals = plsc.load_gather(rows[b], [rowi, ci])
                        plsc.store_scatter(rows[b], [rowi, ci], vals * bet)
                pltpu.async_copy(
                    rows[b], accz.at[esbuf.at[k]], ssems[b], add=True)

                @pl.when(k >= la)
                def _():
                    pltpu.make_async_copy(
                        rows[bp], accz.at[esbuf.at[k - la]], ssems[bp]).wait()

                @pl.when(k + la < _CHT)
                def _():
                    pltpu.async_copy(
                        mu_sh.at[vbuf.at[k + la]], rows[bp], gsems[bp])

        for k in range(_CHT - la, _CHT):
            pltpu.make_async_copy(
                rows[k % rz], accz.at[esbuf.at[k]], ssems[k % rz]).wait()

        plsc.subcore_barrier()
        pltpu.sync_copy(betatile, beta_h.at[pl.ds(wid * _PT, _PT)])

        @pl.when(s == 0)
        def _():
            pltpu.sync_copy(accz, zp_h.at[c])

    return kfn(ex, rden, mu, v_g, e_g, e_s, zz)


def _mm_relu(x, w, b, block_rows=1000):
    n, k = x.shape
    _, m = w.shape

    def body(x_ref, w_ref, b_ref, o_ref):
        acc = jnp.dot(x_ref[...], w_ref[...], preferred_element_type=jnp.float32)
        o_ref[...] = jax.nn.relu(acc + b_ref[...])

    return pl.pallas_call(
        body,
        grid=(n // block_rows,),
        in_specs=[
            pl.BlockSpec((block_rows, k), lambda i: (i, 0)),
            pl.BlockSpec((k, m), lambda i: (0, 0)),
            pl.BlockSpec((1, m), lambda i: (0, 0)),
        ],
        out_specs=pl.BlockSpec((block_rows, m), lambda i: (i, 0)),
        out_shape=jax.ShapeDtypeStruct((n, m), jnp.float32),
    )(x, w, b.reshape(1, m))


def _deg_project(dvp, dep, h0, wc, block_rows=1000):
    n = h0.shape[0]
    cin = h0.shape[1]
    m = wc.shape[1]

    def body(dv_ref, de_ref, h_ref, w_ref, x0_ref, dvis_ref, dei_ref):
        i = pl.program_id(0)
        dv = dv_ref[0, :, 0:1] + dv_ref[1, :, 0:1]
        dvis = jnp.where(dv > 0, lax.rsqrt(jnp.maximum(dv, 1e-12)), 0.0)
        dvis_ref[...] = dvis
        x0_ref[...] = jnp.dot(h_ref[...] * dvis, w_ref[...],
                              preferred_element_type=jnp.float32)

        @pl.when(i == 0)
        def _():
            de = de_ref[0, :, 0:1] + de_ref[1, :, 0:1]
            dei_ref[...] = jnp.where(de > 0, 1.0 / jnp.maximum(de, 1e-12), 0.0)

    return pl.pallas_call(
        body,
        grid=(n // block_rows,),
        in_specs=[
            pl.BlockSpec((2, block_rows, _L), lambda i: (0, i, 0)),
            pl.BlockSpec((2, _E, _L), lambda i: (0, 0, 0)),
            pl.BlockSpec((block_rows, cin), lambda i: (i, 0)),
            pl.BlockSpec((cin, m), lambda i: (0, 0)),
        ],
        out_specs=[
            pl.BlockSpec((block_rows, m), lambda i: (i, 0)),
            pl.BlockSpec((block_rows, 1), lambda i: (i, 0)),
            pl.BlockSpec((_E, 1), lambda i: (0, 0)),
        ],
        out_shape=[
            jax.ShapeDtypeStruct((n, m), jnp.float32),
            jax.ShapeDtypeStruct((n, 1), jnp.float32),
            jax.ShapeDtypeStruct((_E, 1), jnp.float32),
        ],
    )(dvp, dep, h0, wc)


def _comb_scale(yp, scale, rows):
    _, _, w = yp.shape

    def body(y_ref, s_ref, o_ref):
        o_ref[...] = (y_ref[0] + y_ref[1]) * s_ref[...]

    return pl.pallas_call(
        body,
        grid=(1,),
        in_specs=[
            pl.BlockSpec((2, rows, w), lambda i: (0, 0, 0)),
            pl.BlockSpec((rows, 1), lambda i: (0, 0)),
        ],
        out_specs=pl.BlockSpec((rows, w), lambda i: (0, 0)),
        out_shape=jax.ShapeDtypeStruct((rows, w), jnp.float32),
    )(yp, scale)


def _mid_project(pp, dv_is, bc, block_rows=1000):
    _, _, w = pp.shape
    n = dv_is.shape[0]

    def body(p_ref, d_ref, b_ref, o_ref):
        d = d_ref[...]
        o_ref[...] = d * d * (p_ref[0] + p_ref[1]) + d * b_ref[...]

    return pl.pallas_call(
        body,
        grid=(n // block_rows,),
        in_specs=[
            pl.BlockSpec((2, block_rows, w), lambda i: (0, i, 0)),
            pl.BlockSpec((block_rows, 1), lambda i: (i, 0)),
            pl.BlockSpec((1, w), lambda i: (0, 0)),
        ],
        out_specs=pl.BlockSpec((block_rows, w), lambda i: (i, 0)),
        out_shape=jax.ShapeDtypeStruct((n, w), jnp.float32),
    )(pp, dv_is, bc.reshape(1, w))


def _final_smooth_attention(rp, dv_is, a1, ab1, a2, c_out, block_rows=1000):
    _, _, w = rp.shape
    n = dv_is.shape[0]
    ah = a1.shape[1]

    def body(r_ref, d_ref, a1_ref, ab1_ref, a2_ref,
             mu_ref, lv_ref, sc_ref, mx_ref):
        i = pl.program_id(0)
        s2 = (r_ref[0] + r_ref[1]) * d_ref[...]
        mu = s2[:, :c_out]
        mu_ref[...] = mu
        lv_ref[...] = s2[:, c_out:]
        t = jnp.tanh(jnp.dot(mu, a1_ref[...], preferred_element_type=jnp.float32)
                     + ab1_ref[...])
        sc = jnp.dot(t, a2_ref[...], preferred_element_type=jnp.float32)
        sc_ref[...] = sc

        @pl.when(i == 0)
        def _():
            mx_ref[...] = jnp.full_like(mx_ref[...], -jnp.inf)
        mx_ref[...] = jnp.maximum(mx_ref[...], jnp.max(sc))

    return pl.pallas_call(
        body,
        grid=(n // block_rows,),
        in_specs=[
            pl.BlockSpec((2, block_rows, w), lambda i: (0, i, 0)),
            pl.BlockSpec((block_rows, 1), lambda i: (i, 0)),
            pl.BlockSpec((c_out, ah), lambda i: (0, 0)),
            pl.BlockSpec((1, ah), lambda i: (0, 0)),
            pl.BlockSpec((ah, 1), lambda i: (0, 0)),
        ],
        out_specs=[
            pl.BlockSpec((block_rows, c_out), lambda i: (i, 0)),
            pl.BlockSpec((block_rows, c_out), lambda i: (i, 0)),
            pl.BlockSpec((block_rows, 1), lambda i: (i, 0)),
            pl.BlockSpec((1, 1), lambda i: (0, 0)),
        ],
        out_shape=[
            jax.ShapeDtypeStruct((n, c_out), jnp.float32),
            jax.ShapeDtypeStruct((n, c_out), jnp.float32),
            jax.ShapeDtypeStruct((n, 1), jnp.float32),
            jax.ShapeDtypeStruct((1, 1), jnp.float32),
        ],
    )(rp, dv_is, a1, ab1.reshape(1, ah), a2)


def _shift_scores(scores, gmax):
    n = scores.shape[0]

    def body(s_ref, m_ref, o_ref):
        o_ref[...] = s_ref[...] - m_ref[0, 0]

    return pl.pallas_call(
        body,
        grid=(1,),
        in_specs=[
            pl.BlockSpec((n, 1), lambda i: (0, 0)),
            pl.BlockSpec((1, 1), lambda i: (0, 0)),
        ],
        out_specs=pl.BlockSpec((n, 1), lambda i: (0, 0)),
        out_shape=jax.ShapeDtypeStruct((n, 1), jnp.float32),
    )(scores, gmax)


def _rden_kernel(denp):
    def body(d_ref, o_ref):
        den = jnp.sum(d_ref[0] + d_ref[1], axis=-1, keepdims=True)
        o_ref[...] = 1.0 / jnp.maximum(den, 1e-12)

    return pl.pallas_call(
        body,
        grid=(1,),
        in_specs=[pl.BlockSpec((2, _E, _L), lambda i: (0, 0, 0))],
        out_specs=pl.BlockSpec((_E, 1), lambda i: (0, 0)),
        out_shape=jax.ShapeDtypeStruct((_E, 1), jnp.float32),
    )(denp)


def _decode(mu, zp, block_rows=1000):
    n, c = mu.shape

    def body(x_ref, z_ref, h_ref, z_out_ref):
        i = pl.program_id(0)
        z = z_ref[0] + z_ref[1]

        @pl.when(i == 0)
        def _():
            z_out_ref[...] = z
        acc = lax.dot_general(
            x_ref[...], z,
            dimension_numbers=(((1,), (1,)), ((), ())),
            preferred_element_type=jnp.float32)
        h_ref[...] = jax.nn.sigmoid(acc)

    return pl.pallas_call(
        body,
        grid=(n // block_rows,),
        in_specs=[
            pl.BlockSpec((block_rows, c), lambda i: (i, 0)),
            pl.BlockSpec((2, _E, c), lambda i: (0, 0, 0)),
        ],
        out_specs=[
            pl.BlockSpec((block_rows, _E), lambda i: (i, 0)),
            pl.BlockSpec((_E, c), lambda i: (0, 0)),
        ],
        out_shape=[
            jax.ShapeDtypeStruct((n, _E), jnp.float32),
            jax.ShapeDtypeStruct((_E, c), jnp.float32),
        ],
    )(mu, zp)


def kernel(X, vertex_idx, hyperedge_idx, W1, b1, W2, b2, W3, b3, A1, ab1, A2):
    c_out = W2.shape[1]
    pad = _NNZP - _NNZ

    vi = vertex_idx.astype(jnp.int32)
    ei = hyperedge_idx.astype(jnp.int32)
    v_g = jnp.concatenate([vi, jnp.zeros((pad,), jnp.int32)]).reshape(_ROWS, _CHA)
    v_s = jnp.concatenate([vi, jnp.full((pad,), _N, jnp.int32)]).reshape(_ROWS, _CHA)
    e_g = jnp.concatenate([ei, jnp.zeros((pad,), jnp.int32)]).reshape(_ROWS, _CHA)
    e_s = jnp.concatenate([ei, jnp.full((pad,), _E, jnp.int32)]).reshape(_ROWS, _CHA)

    ones_tab = jnp.ones((_CHA, _L), jnp.float32)
    z_n16 = jnp.zeros((_N + 1, _L), jnp.float32)
    z_e16 = jnp.zeros((_E + 1, _L), jnp.float32)
    z_e64 = jnp.zeros((_E + 1, 64), jnp.float32)
    z_n64 = jnp.zeros((_N + 1, 64), jnp.float32)
    z_row16 = jnp.zeros((_CHA, _L), jnp.float32)
    z_e32 = jnp.zeros((_E + 1, c_out), jnp.float32)

    dvp, dep = _sc_degrees(v_s, e_s, ones_tab, z_n16, z_e16)
    h0 = _mm_relu(X, W1, b1)

    wc = jnp.concatenate([W2, W3], axis=1)
    bc = jnp.concatenate([b2, b3], axis=0)
    x0, dv_is, de_inv = _deg_project(dvp, dep, h0, wc)

    yp = _sc_pair_pass(x0, v_g, e_s, _E, z_e64)
    ys = _comb_scale(yp, de_inv, _E)
    pp = _sc_pair_pass(ys, e_g, v_s, _N, z_n64)
    gs = _mid_project(pp, dv_is, bc)

    qp = _sc_pair_pass(gs, v_g, e_s, _E, z_e64)
    y2s = _comb_scale(qp, de_inv, _E)
    rp = _sc_pair_pass(y2s, e_g, v_s, _N, z_n64)

    mu, logvar, scores, gmax = _final_smooth_attention(
        rp, dv_is, A1, ab1, A2, c_out)

    ssh = _shift_scores(scores, gmax).reshape(_N)
    ex, denp = _sc_att_ex(ssh, v_g, e_s, z_row16, z_e16)
    rden = _rden_kernel(denp).reshape(_E)
    beta, zpart = _sc_att_z(ex, rden, mu, v_g, e_g, e_s, z_e32)
    h_out, z = _decode(mu, zpart)

    return (mu, z, h_out, mu, logvar, beta[:_NNZ])
